# baseline (speedup 1.0000x reference)
import math
import functools

import jax
import jax.numpy as jnp
from jax import lax
from jax.experimental import pallas as pl
from jax.experimental.pallas import tpu as pltpu
from jax.experimental.pallas import tpu_sc as plsc

_DEPTH = 3
_RATIOS = (2000.0 / 2708.0, 0.5, 0.5)
_NT = 16
_RG = 96
_RH = 128


def _sc_build_gh(src, dst, inv, perm, k0p, np_):
    E = src.shape[0]
    assert E % (_NT * 16) == 0 and k0p % _RG == 0 and np_ % _RH == 0
    assert _RG * np_ == _RH * k0p
    ept = E // _NT
    ncg = k0p // _RG
    nch = np_ // _RH
    dump = _RG * np_
    import functools as _ft

    mesh = plsc.VectorSubcoreMesh(core_axis_name="c", subcore_axis_name="s")

    @_ft.partial(
        pl.kernel,
        mesh=mesh,
        out_type=(
            jax.ShapeDtypeStruct((k0p * np_,), jnp.float32),
            jax.ShapeDtypeStruct((np_ * k0p,), jnp.float32),
        ),
        scratch_types=[
            pltpu.VMEM((ept,), jnp.int32),
            pltpu.VMEM((ept,), jnp.int32),
            pltpu.VMEM((np_,), jnp.int32),
            pltpu.VMEM((k0p,), jnp.int32),
            pltpu.VMEM((np_,), jnp.float32),
            pltpu.VMEM((16,), jnp.float32),
            pltpu.VMEM((ept + 16,), jnp.int32),
            pltpu.VMEM_SHARED((_RG * np_ + 16,), jnp.float32),
        ],
        compiler_params=pltpu.CompilerParams(needs_layout_passes=False),
    )
    def built(src_h, dst_h, inv_h, perm_h, zeros_h, ones_h, gp_h, h0_h,
              src_v, dst_v, inv_v, perm_v, zeros_v, ones_v, list_v, spm):
        core = lax.axis_index("c")
        tile = lax.axis_index("s")
        pltpu.sync_copy(src_h.at[pl.ds(tile * ept, ept)], src_v)
        pltpu.sync_copy(dst_h.at[pl.ds(tile * ept, ept)], dst_v)
        pltpu.sync_copy(inv_h, inv_v)
        pltpu.sync_copy(perm_h, perm_v)
        pltpu.sync_copy(zeros_h, zeros_v)
        pltpu.sync_copy(ones_h, ones_v)
        iota16 = jnp.arange(16, dtype=jnp.int32)

        def one_matrix(nchunks, rows, cols, out_ref, rowinv, diag):
            srows = rows // _NT
            def chunk_body(c, carry):
                @pl.when(c % 2 == core)
                def _chunk():
                    row0 = c * rows

                    def zrow(r, cy):
                        base = (tile * srows + r) * cols
                        pltpu.sync_copy(zeros_v.at[pl.ds(0, cols)],
                                        spm.at[pl.ds(base, cols)])
                        return cy
                    lax.fori_loop(0, srows, zrow, 0)

                    @pl.when(tile == 0)
                    def _zdump():
                        pltpu.sync_copy(zeros_v.at[pl.ds(0, 16)],
                                        spm.at[pl.ds(_RG * np_, 16)])
                    plsc.subcore_barrier()

                    if diag:
                        @pl.when(tile < rows // 16)
                        def _diag():
                            a16 = row0 + tile * 16 + iota16
                            p16 = plsc.load_gather(perm_v, [a16])
                            fl = jnp.where(p16 >= 0,
                                           (a16 - row0) * cols + p16, dump)
                            pltpu.sync_copy(ones_v, spm.at[fl], add=True)
                            pltpu.sync_copy(ones_v, spm.at[fl], add=True)

                    def scan(g, cnt):
                        s16 = src_v[pl.ds(g * 16, 16)]
                        d16 = dst_v[pl.ds(g * 16, 16)]
                        if rowinv:
                            row = plsc.load_gather(inv_v, [s16])
                            col = d16
                            m = (row >= row0) & (row < row0 + rows) & (s16 != d16)
                        else:
                            row = s16
                            col = plsc.load_gather(inv_v, [d16])
                            m = ((row >= row0) & (row < row0 + rows)
                                 & (col >= 0) & (s16 != d16))
                        fl = (row - row0) * cols + col
                        plsc.store_compressed(list_v.at[pl.ds(cnt, 16)], fl,
                                              mask=m)
                        return cnt + jnp.sum(m.astype(jnp.int32))
                    cnt = lax.fori_loop(0, ept // 16, scan, 0)
                    list_v[pl.ds(cnt, 16)] = jnp.full((16,), dump, jnp.int32)

                    def scat(j, cy):
                        idx16 = list_v[pl.ds(j * 16, 16)]
                        pltpu.sync_copy(ones_v, spm.at[idx16], add=True)
                        return cy
                    lax.fori_loop(0, (cnt + 15) // 16, scat, 0)
                    plsc.subcore_barrier()

                    sz = srows * cols
                    pltpu.sync_copy(
                        spm.at[pl.ds(tile * sz, sz)],
                        out_ref.at[pl.ds(row0 * cols + tile * sz, sz)])
                return carry
            lax.fori_loop(0, nchunks, chunk_body, 0)

        one_matrix(ncg, _RG, np_, gp_h, rowinv=True, diag=True)
        one_matrix(nch, _RH, k0p, h0_h, rowinv=False, diag=False)

    zeros = jnp.zeros((np_,), jnp.float32)
    ones = jnp.ones((16,), jnp.float32)
    gp, h0 = built(src, dst, inv, perm, zeros, ones)
    return gp.reshape(k0p, np_), h0.reshape(np_, k0p)


def _sc_deg(srcp, dstp, E, np_):
    EP = srcp.shape[0]
    ept = EP // 32
    assert ept % 16 == 0
    mesh = plsc.VectorSubcoreMesh(core_axis_name="c", subcore_axis_name="s")

    @functools.partial(
        pl.kernel,
        mesh=mesh,
        out_type=jax.ShapeDtypeStruct((2, 2 * np_), jnp.float32),
        scratch_types=[
            pltpu.VMEM((ept,), jnp.int32),
            pltpu.VMEM((ept,), jnp.int32),
            pltpu.VMEM((np_,), jnp.float32),
            pltpu.VMEM((16,), jnp.float32),
            pltpu.VMEM_SHARED((2 * np_ + 16,), jnp.float32),
        ],
        compiler_params=pltpu.CompilerParams(needs_layout_passes=False),
    )
    def built(src_h, dst_h, zeros_h, ones_h, out_h, src_v, dst_v, zeros_v,
              ones_v, spm):
        core = lax.axis_index("c")
        tile = lax.axis_index("s")
        base = (core * _NT + tile) * ept
        pltpu.sync_copy(src_h.at[pl.ds(base, ept)], src_v)
        pltpu.sync_copy(dst_h.at[pl.ds(base, ept)], dst_v)
        pltpu.sync_copy(zeros_h, zeros_v)
        pltpu.sync_copy(ones_h, ones_v)
        iota16 = jnp.arange(16, dtype=jnp.int32)
        dump = 2 * np_

        @pl.when(tile < 2)
        def _z():
            pltpu.sync_copy(zeros_v, spm.at[pl.ds(tile * np_, np_)])

        @pl.when(tile == 2)
        def _z2():
            pltpu.sync_copy(zeros_v.at[pl.ds(0, 16)], spm.at[pl.ds(dump, 16)])
        plsc.subcore_barrier()

        def scan(g, cy):
            s16 = src_v[pl.ds(g * 16, 16)]
            d16 = dst_v[pl.ds(g * 16, 16)]
            mv = base + g * 16 + iota16 < E
            pltpu.sync_copy(ones_v, spm.at[jnp.where(mv, s16, dump)], add=True)
            selfm = mv & (s16 == d16)
            pltpu.sync_copy(ones_v, spm.at[jnp.where(selfm, np_ + s16, dump)],
                            add=True)
            return cy
        lax.fori_loop(0, ept // 16, scan, 0)
        plsc.subcore_barrier()

        @pl.when(tile < 2)
        def _wb():
            pltpu.sync_copy(spm.at[pl.ds(tile * np_, np_)],
                            out_h.at[core].at[pl.ds(tile * np_, np_)])

    zeros = jnp.zeros((np_,), jnp.float32)
    ones = jnp.ones((16,), jnp.float32)
    out = built(srcp, dstp, zeros, ones)
    comb = out[0] + out[1]
    return comb[:np_], comb[np_:]


def _sc_msg_chunked(src, dst, u2, np_, nch, nr):
    E = src.shape[0]
    ept = E // _NT
    rchunk = np_ // nr
    assert ept % 16 == 0 and rchunk % _NT == 0 and (rchunk + 16) % _NT == 0
    zrows = (rchunk + 16) // _NT
    mesh = plsc.VectorSubcoreMesh(core_axis_name="c", subcore_axis_name="s")

    @functools.partial(
        pl.kernel,
        mesh=mesh,
        out_type=jax.ShapeDtypeStruct((nch * np_, 128), jnp.float32),
        scratch_types=[
            pltpu.VMEM((ept,), jnp.int32),
            pltpu.VMEM((ept,), jnp.int32),
            pltpu.VMEM((zrows, 128), jnp.float32),
            pltpu.VMEM((16, 128), jnp.float32),
            pltpu.SemaphoreType.DMA,
            pltpu.VMEM_SHARED((rchunk + 16, 128), jnp.float32),
        ],
        compiler_params=pltpu.CompilerParams(needs_layout_passes=False),
    )
    def built(src_h, dst_h, zeros_h, u_h, out_h, src_v, dst_v, zeros_v,
              rows_v, sem, spm):
        core = lax.axis_index("c")
        tile = lax.axis_index("s")
        pltpu.sync_copy(src_h.at[pl.ds(tile * ept, ept)], src_v)
        pltpu.sync_copy(dst_h.at[pl.ds(tile * ept, ept)], dst_v)
        pltpu.sync_copy(zeros_h, zeros_v)

        def chunk(ci, carry):
            @pl.when(ci % 2 == core)
            def _chunk():
                cch = ci // nr
                r0 = (ci % nr) * rchunk
                pltpu.sync_copy(zeros_v, spm.at[pl.ds(tile * zrows, zrows)])
                plsc.subcore_barrier()

                def scan(g, cy):
                    s16 = src_v[pl.ds(g * 16, 16)]
                    d16 = dst_v[pl.ds(g * 16, 16)]
                    ok = ((s16 != d16) & (s16 >= r0)
                          & (s16 < r0 + rchunk))

                    @pl.when(jnp.any(ok))
                    def _go():
                        pltpu.async_copy(
                            u_h.at[cch * np_ + jnp.where(ok, d16, 0)],
                            rows_v, sem).wait()
                        pltpu.sync_copy(
                            rows_v,
                            spm.at[jnp.where(ok, s16 - r0, rchunk)],
                            add=True)
                    return cy
                lax.fori_loop(0, ept // 16, scan, 0)
                plsc.subcore_barrier()

                wb = rchunk // _NT
                pltpu.sync_copy(
                    spm.at[pl.ds(tile * wb, wb)],
                    out_h.at[pl.ds(cch * np_ + r0 + tile * wb, wb)])
            return carry
        lax.fori_loop(0, nch * nr, chunk, 0)

    zeros = jnp.zeros((zrows, 128), jnp.float32)
    return built(src, dst, zeros, u2)


def _ceil_to(v, m):
    return ((v + m - 1) // m) * m


def _pick(dim, cands):
    for c in cands:
        if dim % c == 0:
            return c
    raise AssertionError(dim)


def _mm(a, b, *, zero_diag=False, out_dtype=jnp.float32, split_b=False,
        bn_cap=4096):
    M, K = a.shape
    K2, N = b.shape
    assert K == K2, (a.shape, b.shape)
    bm = _pick(M, (1536, 768, 512, 256, 128))
    bn = _pick(N, (1536, 768, 512, 256, 128))
    bn = min(bn, bn_cap)
    bk = _pick(K, (512, 768, 256, 128))
    nk = K // bk

    def kern(a_ref, b_ref, o_ref, acc_ref):
        @pl.when(pl.program_id(2) == 0)
        def _():
            acc_ref[...] = jnp.zeros_like(acc_ref)

        av = a_ref[...].astype(jnp.bfloat16)
        bv = b_ref[...]
        if split_b:
            bhi = bv.astype(jnp.bfloat16)
            blo = (bv - bhi.astype(jnp.float32)).astype(jnp.bfloat16)
            acc_ref[...] += (jnp.dot(av, bhi, preferred_element_type=jnp.float32)
                             + jnp.dot(av, blo, preferred_element_type=jnp.float32))
        else:
            acc_ref[...] += jnp.dot(av, bv.astype(jnp.bfloat16),
                                    preferred_element_type=jnp.float32)

        if zero_diag:
            gi = pl.program_id(0) * bm + lax.broadcasted_iota(jnp.int32, (bm, bn), 0)
            gj = pl.program_id(1) * bn + lax.broadcasted_iota(jnp.int32, (bm, bn), 1)
            diag = gi == gj
        else:
            diag = None
        last = pl.program_id(2) == nk - 1

        @pl.when(last)
        def _():
            r = acc_ref[...]
            if diag is not None:
                r = jnp.where(diag, 0.0, r)
            o_ref[...] = r.astype(out_dtype)

    return pl.pallas_call(
        kern,
        grid=(M // bm, N // bn, nk),
        in_specs=[
            pl.BlockSpec((bm, bk), lambda i, j, k: (i, k)),
            pl.BlockSpec((bk, bn), lambda i, j, k: (k, j)),
        ],
        out_specs=pl.BlockSpec((bm, bn), lambda i, j, k: (i, j)),
        out_shape=jax.ShapeDtypeStruct((M, N), out_dtype),
        scratch_shapes=[pltpu.VMEM((bm, bn), jnp.float32)],
        compiler_params=pltpu.CompilerParams(
            dimension_semantics=("parallel", "parallel", "arbitrary")),
    )(a, b)


def _mm_f32(a, b):
    M, K = a.shape
    _, N = b.shape
    bm = _pick(M, (512, 256, 128))
    bn = _pick(N, (256, 128))
    bk = _pick(K, (512, 256, 128, 32))
    nk = K // bk

    def kern(a_ref, b_ref, o_ref, acc_ref):
        @pl.when(pl.program_id(2) == 0)
        def _():
            acc_ref[...] = jnp.zeros_like(acc_ref)

        acc_ref[...] += jnp.dot(a_ref[...], b_ref[...],
                                preferred_element_type=jnp.float32,
                                precision=lax.Precision.HIGHEST)
        last = pl.program_id(2) == nk - 1

        @pl.when(last)
        def _():
            o_ref[...] = acc_ref[...]

    return pl.pallas_call(
        kern,
        grid=(M // bm, N // bn, nk),
        in_specs=[
            pl.BlockSpec((bm, bk), lambda i, j, k: (i, k)),
            pl.BlockSpec((bk, bn), lambda i, j, k: (k, j)),
        ],
        out_specs=pl.BlockSpec((bm, bn), lambda i, j, k: (i, j)),
        out_shape=jax.ShapeDtypeStruct((M, N), jnp.float32),
        scratch_shapes=[pltpu.VMEM((bm, bn), jnp.float32)],
        compiler_params=pltpu.CompilerParams(
            dimension_semantics=("parallel", "parallel", "arbitrary")),
    )(a, b)


def _dis_from_rowsum(Al):
    M, N = Al.shape
    bm = _pick(M, (768, 512, 256, 128))
    bn = _pick(N, (768, 512, 256, 128))
    nj = N // bn

    def kern(a_ref, o_ref, acc_ref):
        @pl.when(pl.program_id(1) == 0)
        def _():
            acc_ref[...] = jnp.zeros_like(acc_ref)

        acc_ref[...] += jnp.sum(a_ref[...].astype(jnp.float32), axis=1,
                                keepdims=True)

        @pl.when(pl.program_id(1) == nj - 1)
        def _():
            o_ref[...] = lax.rsqrt(acc_ref[...] + 2.0)

    return pl.pallas_call(
        kern,
        grid=(M // bm, nj),
        in_specs=[pl.BlockSpec((bm, bn), lambda i, j: (i, j))],
        out_specs=pl.BlockSpec((bm, 1), lambda i, j: (i, 0)),
        out_shape=jax.ShapeDtypeStruct((M, 1), jnp.float32),
        scratch_shapes=[pltpu.VMEM((bm, 1), jnp.float32)],
        compiler_params=pltpu.CompilerParams(
            dimension_semantics=("parallel", "arbitrary")),
    )(Al)


def _pad2(a, m, n):
    return jnp.pad(a, ((0, m - a.shape[0]), (0, n - a.shape[1])))


def _mm_feat(a, w):
    M = a.shape[0]
    K = _ceil_to(a.shape[1], 32)
    N = w.shape[1]
    Np = _ceil_to(N, 128)
    ap = _pad2(a, M, K)
    wp = _pad2(w, K, Np)
    r = _mm_f32(ap, wp)
    return r[:, :N]


def _gcn_dense(Al, xl, W, b):
    dis = _dis_from_rowsum(Al)
    xw = _mm_feat(xl, W)
    u = dis * xw
    kp = Al.shape[0]
    up = _pad2(u, kp, 128)
    m = _mm(Al, up, split_b=True)[:, : xw.shape[1]]
    out = dis * m + (2.0 * dis * dis) * xw + b[None, :]
    return out


_RATIOS_K = {}


def _pool(xl, w, nreal):
    nl = xl.shape[0]
    k = int(math.ceil(_RATIOS_K[nreal]))
    score = jnp.tanh((xl @ w) / jnp.linalg.norm(w))
    score = jnp.where(jnp.arange(nl) < nreal, score, -2.0)
    _, permd = lax.top_k(score, k)
    perm = jnp.sort(permd)
    sv = score[perm]
    xp = xl[perm] * sv[:, None]
    inv = jnp.full((nl,), -1, jnp.int32).at[perm].set(jnp.arange(k, dtype=jnp.int32))
    return perm, inv, xp, k


def kernel(x, edge_index, Wd0, bd0, Wd1, bd1, Wd2, bd2, Wd3, bd3,
           p0, p1, p2, Wu0, bu0, Wu1, bu1, Wu2, bu2):
    n = x.shape[0]
    hid = Wd0.shape[1]
    src, dst = edge_index[0], edge_index[1]
    selfm = src == dst

    k0 = int(math.ceil(_RATIOS[0] * n))
    k1 = int(math.ceil(_RATIOS[1] * k0))
    k2 = int(math.ceil(_RATIOS[2] * k1))
    np_ = _ceil_to(n, 512)
    k0p = _ceil_to(k0, 768 if k0 > 768 else 256)
    k1p = _ceil_to(k1, 768 if k1 > 768 else 256)
    k2p = _ceil_to(k2, 768 if k2 > 768 else 256)
    _RATIOS_K[n] = _RATIOS[0] * n
    _RATIOS_K[k0] = _RATIOS[1] * k0
    _RATIOS_K[k1] = _RATIOS[2] * k1

    E = src.shape[0]
    big = n >= 1024
    if big:
        EP2 = _ceil_to(E, 512)
        srcp = jnp.pad(src, (0, EP2 - E))
        dstp = jnp.pad(dst, (0, EP2 - E))
        rowsum0, c0 = _sc_deg(srcp, dstp, E, np_)
        rowsum0, c0 = rowsum0[:n], c0[:n]
    else:
        ones_e = jnp.ones(src.shape, jnp.float32)
        rowsum0 = jnp.zeros((n,), jnp.float32).at[src].add(ones_e)
        c0 = jnp.zeros((n,), jnp.float32).at[src].add(
            jnp.where(selfm, 1.0, 0.0))
    a2ii = jnp.where(c0 > 0, c0, 2.0)
    deg0 = rowsum0 - c0 + a2ii
    dis0 = lax.rsqrt(deg0)

    def gcn0(xl, W, b):
        xw = _mm_feat(_pad2(xl, np_, xl.shape[1]), W)[:n]
        C = W.shape[1]
        u = dis0[:, None] * xw
        if not big:
            msg = jnp.zeros_like(u).at[src].add(
                jnp.where(selfm, 0.0, 1.0)[:, None] * u[dst])
        else:
            nch = max(1, C // 128)
            Cp = nch * 128
            u2 = _pad2(u, np_, Cp).reshape(np_, nch, 128)
            u2 = u2.transpose(1, 0, 2).reshape(nch * np_, 128)
            m2 = _sc_msg_chunked(src, dst, u2, np_, nch, 2)
            msg = m2.reshape(nch, np_, 128).transpose(1, 0, 2)
            msg = msg.reshape(np_, Cp)[:n, :C]
        return dis0[:, None] * msg + (a2ii * dis0 * dis0)[:, None] * xw + b[None, :]

    x1 = jax.nn.relu(gcn0(x, Wd0, bd0))

    x1p = _pad2(x1, np_, hid)
    perm0, inv0, xp0, _ = _pool(x1p, p0, n)
    if n >= 1024:
        perm0p = jnp.pad(perm0, (0, k0p - k0), constant_values=-1)
        Gp, H0 = _sc_build_gh(src, dst, inv0, perm0p, k0p, np_)
    else:
        one = jnp.ones((), jnp.float32)
        gi = jnp.where((inv0[src] >= 0) & (~selfm), inv0[src], k0p)
        Gp = jnp.zeros((k0p, np_), jnp.float32).at[gi, dst].add(one, mode="drop")
        Gp = Gp.at[jnp.arange(k0), perm0[:k0]].add(2.0, mode="drop")
        hi = jnp.where((inv0[dst] >= 0) & (~selfm), inv0[dst], k0p)
        H0 = jnp.zeros((np_, k0p), jnp.float32).at[src, hi].add(one, mode="drop")
    A1 = _mm(Gp, H0, zero_diag=True, out_dtype=jnp.bfloat16)

    xp0 = _pad2(xp0[:k0], k0p, hid)
    x2 = jax.nn.relu(_gcn_dense(A1, xp0, Wd1, bd1))

    perm1, inv1, xp1, _ = _pool(x2, p1, k0)
    G1 = jnp.pad(A1[perm1[:k1]], ((0, k1p - k1), (0, 0)))
    G1 = G1.at[jnp.arange(k1), perm1[:k1]].add(jnp.bfloat16(2.0))
    H1 = jnp.pad(A1[:, perm1[:k1]], ((0, 0), (0, k1p - k1)))
    A2 = _mm(G1, H1, zero_diag=True, out_dtype=jnp.bfloat16)

    xp1 = _pad2(xp1[:k1], k1p, hid)
    x3 = jax.nn.relu(_gcn_dense(A2, xp1, Wd2, bd2))

    perm2, inv2, xp2, _ = _pool(x3, p2, k1)
    G2 = jnp.pad(A2[perm2[:k2]], ((0, k2p - k2), (0, 0)))
    G2 = G2.at[jnp.arange(k2), perm2[:k2]].add(jnp.bfloat16(2.0))
    H2 = jnp.pad(A2[:, perm2[:k2]], ((0, 0), (0, k2p - k2)))
    A3 = _mm(G2, H2, zero_diag=True, out_dtype=jnp.bfloat16)

    xp2 = _pad2(xp2[:k2], k2p, hid)
    x4 = jax.nn.relu(_gcn_dense(A3, xp2, Wd3, bd3))

    up = jnp.zeros_like(x3).at[perm2[:k2]].set(x4[:k2])
    xq = x3 + up
    xq = jax.nn.relu(_gcn_dense(A2, xq, Wu0, bu0))
    up = jnp.zeros_like(x2).at[perm1[:k1]].set(xq[:k1])
    xq = x2 + up
    xq = jax.nn.relu(_gcn_dense(A1, xq, Wu1, bu1))
    up = jnp.zeros((n, hid), jnp.float32).at[perm0].set(xq[:k0], mode="drop")
    xq = x1 + up
    xq = gcn0(xq, Wu2, bu2)
    return jax.nn.log_softmax(xq, axis=1)


# speedup vs baseline: 3.1298x; 3.1298x over previous
import math
import functools

import jax
import jax.numpy as jnp
from jax import lax
from jax.experimental import pallas as pl
from jax.experimental.pallas import tpu as pltpu
from jax.experimental.pallas import tpu_sc as plsc

_DEPTH = 3
_RATIOS = (2000.0 / 2708.0, 0.5, 0.5)
_NT = 16
_RG = 96
_RH = 128


def _sc_build_gh(src, dst, inv, perm, k0p, np_):
    E = src.shape[0]
    assert E % (_NT * 16) == 0 and k0p % _RG == 0 and np_ % _RH == 0
    assert _RG * np_ == _RH * k0p
    ept = E // _NT
    ncg = k0p // _RG
    nch = np_ // _RH
    dump = _RG * np_
    import functools as _ft

    mesh = plsc.VectorSubcoreMesh(core_axis_name="c", subcore_axis_name="s")

    @_ft.partial(
        pl.kernel,
        mesh=mesh,
        out_type=(
            jax.ShapeDtypeStruct((k0p * np_,), jnp.float32),
            jax.ShapeDtypeStruct((np_ * k0p,), jnp.float32),
        ),
        scratch_types=[
            pltpu.VMEM((ept,), jnp.int32),
            pltpu.VMEM((ept,), jnp.int32),
            pltpu.VMEM((np_,), jnp.int32),
            pltpu.VMEM((k0p,), jnp.int32),
            pltpu.VMEM((np_,), jnp.float32),
            pltpu.VMEM((16,), jnp.float32),
            pltpu.VMEM((ept + 16,), jnp.int32),
            pltpu.VMEM_SHARED((_RG * np_ + 16,), jnp.float32),
        ],
        compiler_params=pltpu.CompilerParams(needs_layout_passes=False),
    )
    def built(src_h, dst_h, inv_h, perm_h, zeros_h, ones_h, gp_h, h0_h,
              src_v, dst_v, inv_v, perm_v, zeros_v, ones_v, list_v, spm):
        core = lax.axis_index("c")
        tile = lax.axis_index("s")
        pltpu.sync_copy(src_h.at[pl.ds(tile * ept, ept)], src_v)
        pltpu.sync_copy(dst_h.at[pl.ds(tile * ept, ept)], dst_v)
        pltpu.sync_copy(inv_h, inv_v)
        pltpu.sync_copy(perm_h, perm_v)
        pltpu.sync_copy(zeros_h, zeros_v)
        pltpu.sync_copy(ones_h, ones_v)
        iota16 = jnp.arange(16, dtype=jnp.int32)

        def one_matrix(nchunks, rows, cols, out_ref, rowinv, diag):
            srows = rows // _NT
            def chunk_body(c, carry):
                @pl.when(c % 2 == core)
                def _chunk():
                    row0 = c * rows

                    def zrow(r, cy):
                        base = (tile * srows + r) * cols
                        pltpu.sync_copy(zeros_v.at[pl.ds(0, cols)],
                                        spm.at[pl.ds(base, cols)])
                        return cy
                    lax.fori_loop(0, srows, zrow, 0)

                    @pl.when(tile == 0)
                    def _zdump():
                        pltpu.sync_copy(zeros_v.at[pl.ds(0, 16)],
                                        spm.at[pl.ds(_RG * np_, 16)])
                    plsc.subcore_barrier()

                    if diag:
                        @pl.when(tile < rows // 16)
                        def _diag():
                            a16 = row0 + tile * 16 + iota16
                            p16 = plsc.load_gather(perm_v, [a16])
                            fl = jnp.where(p16 >= 0,
                                           (a16 - row0) * cols + p16, dump)
                            pltpu.sync_copy(ones_v, spm.at[fl], add=True)
                            pltpu.sync_copy(ones_v, spm.at[fl], add=True)

                    def scan(g, cnt):
                        s16 = src_v[pl.ds(g * 16, 16)]
                        d16 = dst_v[pl.ds(g * 16, 16)]
                        if rowinv:
                            row = plsc.load_gather(inv_v, [s16])
                            col = d16
                            m = (row >= row0) & (row < row0 + rows) & (s16 != d16)
                        else:
                            row = s16
                            col = plsc.load_gather(inv_v, [d16])
                            m = ((row >= row0) & (row < row0 + rows)
                                 & (col >= 0) & (s16 != d16))
                        fl = (row - row0) * cols + col
                        plsc.store_compressed(list_v.at[pl.ds(cnt, 16)], fl,
                                              mask=m)
                        return cnt + jnp.sum(m.astype(jnp.int32))
                    cnt = lax.fori_loop(0, ept // 16, scan, 0)
                    list_v[pl.ds(cnt, 16)] = jnp.full((16,), dump, jnp.int32)

                    def scat(j, cy):
                        idx16 = list_v[pl.ds(j * 16, 16)]
                        pltpu.sync_copy(ones_v, spm.at[idx16], add=True)
                        return cy
                    lax.fori_loop(0, (cnt + 15) // 16, scat, 0)
                    plsc.subcore_barrier()

                    sz = srows * cols
                    pltpu.sync_copy(
                        spm.at[pl.ds(tile * sz, sz)],
                        out_ref.at[pl.ds(row0 * cols + tile * sz, sz)])
                return carry
            lax.fori_loop(0, nchunks, chunk_body, 0)

        one_matrix(ncg, _RG, np_, gp_h, rowinv=True, diag=True)
        one_matrix(nch, _RH, k0p, h0_h, rowinv=False, diag=False)

    zeros = jnp.zeros((np_,), jnp.float32)
    ones = jnp.ones((16,), jnp.float32)
    gp, h0 = built(src, dst, inv, perm, zeros, ones)
    return gp.reshape(k0p, np_), h0.reshape(np_, k0p)


def _sc_deg(srcp, dstp, E, np_):
    EP = srcp.shape[0]
    ept = EP // 32
    assert ept % 16 == 0
    mesh = plsc.VectorSubcoreMesh(core_axis_name="c", subcore_axis_name="s")

    @functools.partial(
        pl.kernel,
        mesh=mesh,
        out_type=jax.ShapeDtypeStruct((2, 2 * np_), jnp.float32),
        scratch_types=[
            pltpu.VMEM((ept,), jnp.int32),
            pltpu.VMEM((ept,), jnp.int32),
            pltpu.VMEM((np_,), jnp.float32),
            pltpu.VMEM((16,), jnp.float32),
            pltpu.VMEM_SHARED((2 * np_ + 16,), jnp.float32),
        ],
        compiler_params=pltpu.CompilerParams(needs_layout_passes=False),
    )
    def built(src_h, dst_h, zeros_h, ones_h, out_h, src_v, dst_v, zeros_v,
              ones_v, spm):
        core = lax.axis_index("c")
        tile = lax.axis_index("s")
        base = (core * _NT + tile) * ept
        pltpu.sync_copy(src_h.at[pl.ds(base, ept)], src_v)
        pltpu.sync_copy(dst_h.at[pl.ds(base, ept)], dst_v)
        pltpu.sync_copy(zeros_h, zeros_v)
        pltpu.sync_copy(ones_h, ones_v)
        iota16 = jnp.arange(16, dtype=jnp.int32)
        dump = 2 * np_

        @pl.when(tile < 2)
        def _z():
            pltpu.sync_copy(zeros_v, spm.at[pl.ds(tile * np_, np_)])

        @pl.when(tile == 2)
        def _z2():
            pltpu.sync_copy(zeros_v.at[pl.ds(0, 16)], spm.at[pl.ds(dump, 16)])
        plsc.subcore_barrier()

        def scan(g, cy):
            s16 = src_v[pl.ds(g * 16, 16)]
            d16 = dst_v[pl.ds(g * 16, 16)]
            mv = base + g * 16 + iota16 < E
            pltpu.sync_copy(ones_v, spm.at[jnp.where(mv, s16, dump)], add=True)
            selfm = mv & (s16 == d16)
            pltpu.sync_copy(ones_v, spm.at[jnp.where(selfm, np_ + s16, dump)],
                            add=True)
            return cy
        lax.fori_loop(0, ept // 16, scan, 0)
        plsc.subcore_barrier()

        @pl.when(tile < 2)
        def _wb():
            pltpu.sync_copy(spm.at[pl.ds(tile * np_, np_)],
                            out_h.at[core].at[pl.ds(tile * np_, np_)])

    zeros = jnp.zeros((np_,), jnp.float32)
    ones = jnp.ones((16,), jnp.float32)
    out = built(srcp, dstp, zeros, ones)
    comb = out[0] + out[1]
    return comb[:np_], comb[np_:]


def _sc_msg_chunked(src, dst, u2, np_, nch, nr):
    E = src.shape[0]
    ept = E // _NT
    rchunk = np_ // nr
    assert ept % 16 == 0 and rchunk % _NT == 0 and (rchunk + 16) % _NT == 0
    zrows = (rchunk + 16) // _NT
    mesh = plsc.VectorSubcoreMesh(core_axis_name="c", subcore_axis_name="s")

    @functools.partial(
        pl.kernel,
        mesh=mesh,
        out_type=jax.ShapeDtypeStruct((nch * np_, 128), jnp.float32),
        scratch_types=[
            pltpu.VMEM((ept,), jnp.int32),
            pltpu.VMEM((ept,), jnp.int32),
            pltpu.VMEM((zrows, 128), jnp.float32),
            pltpu.VMEM((16, 128), jnp.float32),
            pltpu.SemaphoreType.DMA,
            pltpu.VMEM_SHARED((rchunk + 16, 128), jnp.float32),
        ],
        compiler_params=pltpu.CompilerParams(needs_layout_passes=False),
    )
    def built(src_h, dst_h, zeros_h, u_h, out_h, src_v, dst_v, zeros_v,
              rows_v, sem, spm):
        core = lax.axis_index("c")
        tile = lax.axis_index("s")
        pltpu.sync_copy(src_h.at[pl.ds(tile * ept, ept)], src_v)
        pltpu.sync_copy(dst_h.at[pl.ds(tile * ept, ept)], dst_v)
        pltpu.sync_copy(zeros_h, zeros_v)

        def chunk(ci, carry):
            @pl.when(ci % 2 == core)
            def _chunk():
                cch = ci // nr
                r0 = (ci % nr) * rchunk
                pltpu.sync_copy(zeros_v, spm.at[pl.ds(tile * zrows, zrows)])
                plsc.subcore_barrier()

                def scan(g, cy):
                    s16 = src_v[pl.ds(g * 16, 16)]
                    d16 = dst_v[pl.ds(g * 16, 16)]
                    ok = ((s16 != d16) & (s16 >= r0)
                          & (s16 < r0 + rchunk))

                    @pl.when(jnp.any(ok))
                    def _go():
                        pltpu.async_copy(
                            u_h.at[cch * np_ + jnp.where(ok, d16, 0)],
                            rows_v, sem).wait()
                        pltpu.sync_copy(
                            rows_v,
                            spm.at[jnp.where(ok, s16 - r0, rchunk)],
                            add=True)
                    return cy
                lax.fori_loop(0, ept // 16, scan, 0)
                plsc.subcore_barrier()

                wb = rchunk // _NT
                pltpu.sync_copy(
                    spm.at[pl.ds(tile * wb, wb)],
                    out_h.at[pl.ds(cch * np_ + r0 + tile * wb, wb)])
            return carry
        lax.fori_loop(0, nch * nr, chunk, 0)

    zeros = jnp.zeros((zrows, 128), jnp.float32)
    return built(src, dst, zeros, u2)


def _ceil_to(v, m):
    return ((v + m - 1) // m) * m


def _pick(dim, cands):
    for c in cands:
        if dim % c == 0:
            return c
    raise AssertionError(dim)


def _mm(a, b, *, zero_diag=False, out_dtype=jnp.float32, split_b=False,
        bn_cap=4096):
    M, K = a.shape
    K2, N = b.shape
    assert K == K2, (a.shape, b.shape)
    bm = _pick(M, (1536, 768, 512, 256, 128))
    bn = _pick(N, (1536, 768, 512, 256, 128))
    bn = min(bn, bn_cap)
    bk = _pick(K, (512, 768, 256, 128))
    nk = K // bk

    def kern(a_ref, b_ref, o_ref, acc_ref):
        @pl.when(pl.program_id(2) == 0)
        def _():
            acc_ref[...] = jnp.zeros_like(acc_ref)

        av = a_ref[...].astype(jnp.bfloat16)
        bv = b_ref[...]
        if split_b:
            bhi = bv.astype(jnp.bfloat16)
            blo = (bv - bhi.astype(jnp.float32)).astype(jnp.bfloat16)
            acc_ref[...] += (jnp.dot(av, bhi, preferred_element_type=jnp.float32)
                             + jnp.dot(av, blo, preferred_element_type=jnp.float32))
        else:
            acc_ref[...] += jnp.dot(av, bv.astype(jnp.bfloat16),
                                    preferred_element_type=jnp.float32)

        if zero_diag:
            gi = pl.program_id(0) * bm + lax.broadcasted_iota(jnp.int32, (bm, bn), 0)
            gj = pl.program_id(1) * bn + lax.broadcasted_iota(jnp.int32, (bm, bn), 1)
            diag = gi == gj
        else:
            diag = None
        last = pl.program_id(2) == nk - 1

        @pl.when(last)
        def _():
            r = acc_ref[...]
            if diag is not None:
                r = jnp.where(diag, 0.0, r)
            o_ref[...] = r.astype(out_dtype)

    return pl.pallas_call(
        kern,
        grid=(M // bm, N // bn, nk),
        in_specs=[
            pl.BlockSpec((bm, bk), lambda i, j, k: (i, k)),
            pl.BlockSpec((bk, bn), lambda i, j, k: (k, j)),
        ],
        out_specs=pl.BlockSpec((bm, bn), lambda i, j, k: (i, j)),
        out_shape=jax.ShapeDtypeStruct((M, N), out_dtype),
        scratch_shapes=[pltpu.VMEM((bm, bn), jnp.float32)],
        compiler_params=pltpu.CompilerParams(
            dimension_semantics=("parallel", "parallel", "arbitrary")),
    )(a, b)


def _mm_f32(a, b):
    M, K = a.shape
    _, N = b.shape
    bm = _pick(M, (512, 256, 128))
    bn = _pick(N, (256, 128))
    bk = _pick(K, (512, 256, 128, 32))
    nk = K // bk

    def kern(a_ref, b_ref, o_ref, acc_ref):
        @pl.when(pl.program_id(2) == 0)
        def _():
            acc_ref[...] = jnp.zeros_like(acc_ref)

        acc_ref[...] += jnp.dot(a_ref[...], b_ref[...],
                                preferred_element_type=jnp.float32,
                                precision=lax.Precision.HIGHEST)
        last = pl.program_id(2) == nk - 1

        @pl.when(last)
        def _():
            o_ref[...] = acc_ref[...]

    return pl.pallas_call(
        kern,
        grid=(M // bm, N // bn, nk),
        in_specs=[
            pl.BlockSpec((bm, bk), lambda i, j, k: (i, k)),
            pl.BlockSpec((bk, bn), lambda i, j, k: (k, j)),
        ],
        out_specs=pl.BlockSpec((bm, bn), lambda i, j, k: (i, j)),
        out_shape=jax.ShapeDtypeStruct((M, N), jnp.float32),
        scratch_shapes=[pltpu.VMEM((bm, bn), jnp.float32)],
        compiler_params=pltpu.CompilerParams(
            dimension_semantics=("parallel", "parallel", "arbitrary")),
    )(a, b)


def _dis_from_rowsum(Al):
    M, N = Al.shape
    bm = _pick(M, (768, 512, 256, 128))
    bn = _pick(N, (768, 512, 256, 128))
    nj = N // bn

    def kern(a_ref, o_ref, acc_ref):
        @pl.when(pl.program_id(1) == 0)
        def _():
            acc_ref[...] = jnp.zeros_like(acc_ref)

        acc_ref[...] += jnp.sum(a_ref[...].astype(jnp.float32), axis=1,
                                keepdims=True)

        @pl.when(pl.program_id(1) == nj - 1)
        def _():
            o_ref[...] = lax.rsqrt(acc_ref[...] + 2.0)

    return pl.pallas_call(
        kern,
        grid=(M // bm, nj),
        in_specs=[pl.BlockSpec((bm, bn), lambda i, j: (i, j))],
        out_specs=pl.BlockSpec((bm, 1), lambda i, j: (i, 0)),
        out_shape=jax.ShapeDtypeStruct((M, 1), jnp.float32),
        scratch_shapes=[pltpu.VMEM((bm, 1), jnp.float32)],
        compiler_params=pltpu.CompilerParams(
            dimension_semantics=("parallel", "arbitrary")),
    )(Al)


def _pad2(a, m, n):
    return jnp.pad(a, ((0, m - a.shape[0]), (0, n - a.shape[1])))


def _mm_feat(a, w):
    M = a.shape[0]
    K = _ceil_to(a.shape[1], 32)
    N = w.shape[1]
    Np = _ceil_to(N, 128)
    ap = _pad2(a, M, K)
    wp = _pad2(w, K, Np)
    r = _mm_f32(ap, wp)
    return r[:, :N]


def _gcn_dense(Al, xl, W, b):
    dis = _dis_from_rowsum(Al)
    xw = _mm_feat(xl, W)
    u = dis * xw
    kp = Al.shape[0]
    up = _pad2(u, kp, 128)
    m = _mm(Al, up, split_b=True)[:, : xw.shape[1]]
    out = dis * m + (2.0 * dis * dis) * xw + b[None, :]
    return out


_RATIOS_K = {}


def _pool(xl, w, nreal):
    nl = xl.shape[0]
    k = int(math.ceil(_RATIOS_K[nreal]))
    score = jnp.tanh((xl @ w) / jnp.linalg.norm(w))
    score = jnp.where(jnp.arange(nl) < nreal, score, -2.0)
    _, permd = lax.top_k(score, k)
    perm = jnp.sort(permd)
    sv = score[perm]
    xp = xl[perm] * sv[:, None]
    inv = jnp.full((nl,), -1, jnp.int32).at[perm].set(jnp.arange(k, dtype=jnp.int32))
    return perm, inv, xp, k


def kernel(x, edge_index, Wd0, bd0, Wd1, bd1, Wd2, bd2, Wd3, bd3,
           p0, p1, p2, Wu0, bu0, Wu1, bu1, Wu2, bu2):
    n = x.shape[0]
    hid = Wd0.shape[1]
    src, dst = edge_index[0], edge_index[1]
    selfm = src == dst

    k0 = int(math.ceil(_RATIOS[0] * n))
    k1 = int(math.ceil(_RATIOS[1] * k0))
    k2 = int(math.ceil(_RATIOS[2] * k1))
    np_ = _ceil_to(n, 512)
    k0p = _ceil_to(k0, 768 if k0 > 768 else 256)
    k1p = _ceil_to(k1, 768 if k1 > 768 else 256)
    k2p = _ceil_to(k2, 768 if k2 > 768 else 256)
    _RATIOS_K[n] = _RATIOS[0] * n
    _RATIOS_K[k0] = _RATIOS[1] * k0
    _RATIOS_K[k1] = _RATIOS[2] * k1

    E = src.shape[0]
    big = n >= 1024
    if big:
        EP2 = _ceil_to(E, 512)
        srcp = jnp.pad(src, (0, EP2 - E))
        dstp = jnp.pad(dst, (0, EP2 - E))
        rowsum0, c0 = _sc_deg(srcp, dstp, E, np_)
        rowsum0, c0 = rowsum0[:n], c0[:n]
    else:
        ones_e = jnp.ones(src.shape, jnp.float32)
        rowsum0 = jnp.zeros((n,), jnp.float32).at[src].add(ones_e)
        c0 = jnp.zeros((n,), jnp.float32).at[src].add(
            jnp.where(selfm, 1.0, 0.0))
    a2ii = jnp.where(c0 > 0, c0, 2.0)
    deg0 = rowsum0 - c0 + a2ii
    dis0 = lax.rsqrt(deg0)

    def msgpass(u):
        return jnp.zeros_like(u).at[src].add(
            jnp.where(selfm, 0.0, 1.0)[:, None] * u[dst])

    def gcn0(xl, W, b):
        xw = _mm_feat(_pad2(xl, np_, xl.shape[1]), W)[:n]
        if xl.shape[1] <= W.shape[1]:
            m32 = msgpass(dis0[:, None] * xl)
            msg = _mm_feat(_pad2(m32, np_, m32.shape[1]), W)[:n]
        else:
            msg = msgpass(dis0[:, None] * xw)
        return dis0[:, None] * msg + (a2ii * dis0 * dis0)[:, None] * xw + b[None, :]

    x1 = jax.nn.relu(gcn0(x, Wd0, bd0))

    x1p = _pad2(x1, np_, hid)
    perm0, inv0, xp0, _ = _pool(x1p, p0, n)
    if n >= 1024:
        perm0p = jnp.pad(perm0, (0, k0p - k0), constant_values=-1)
        Gp, H0 = _sc_build_gh(src, dst, inv0, perm0p, k0p, np_)
    else:
        one = jnp.ones((), jnp.float32)
        gi = jnp.where((inv0[src] >= 0) & (~selfm), inv0[src], k0p)
        Gp = jnp.zeros((k0p, np_), jnp.float32).at[gi, dst].add(one, mode="drop")
        Gp = Gp.at[jnp.arange(k0), perm0[:k0]].add(2.0, mode="drop")
        hi = jnp.where((inv0[dst] >= 0) & (~selfm), inv0[dst], k0p)
        H0 = jnp.zeros((np_, k0p), jnp.float32).at[src, hi].add(one, mode="drop")
    A1 = _mm(Gp, H0, zero_diag=True, out_dtype=jnp.bfloat16)

    xp0 = _pad2(xp0[:k0], k0p, hid)
    x2 = jax.nn.relu(_gcn_dense(A1, xp0, Wd1, bd1))

    perm1, inv1, xp1, _ = _pool(x2, p1, k0)
    G1 = jnp.pad(A1[perm1[:k1]], ((0, k1p - k1), (0, 0)))
    G1 = G1.at[jnp.arange(k1), perm1[:k1]].add(jnp.bfloat16(2.0))
    H1 = jnp.pad(A1[:, perm1[:k1]], ((0, 0), (0, k1p - k1)))
    A2 = _mm(G1, H1, zero_diag=True, out_dtype=jnp.bfloat16)

    xp1 = _pad2(xp1[:k1], k1p, hid)
    x3 = jax.nn.relu(_gcn_dense(A2, xp1, Wd2, bd2))

    perm2, inv2, xp2, _ = _pool(x3, p2, k1)
    G2 = jnp.pad(A2[perm2[:k2]], ((0, k2p - k2), (0, 0)))
    G2 = G2.at[jnp.arange(k2), perm2[:k2]].add(jnp.bfloat16(2.0))
    H2 = jnp.pad(A2[:, perm2[:k2]], ((0, 0), (0, k2p - k2)))
    A3 = _mm(G2, H2, zero_diag=True, out_dtype=jnp.bfloat16)

    xp2 = _pad2(xp2[:k2], k2p, hid)
    x4 = jax.nn.relu(_gcn_dense(A3, xp2, Wd3, bd3))

    up = jnp.zeros_like(x3).at[perm2[:k2]].set(x4[:k2])
    xq = x3 + up
    xq = jax.nn.relu(_gcn_dense(A2, xq, Wu0, bu0))
    up = jnp.zeros_like(x2).at[perm1[:k1]].set(xq[:k1])
    xq = x2 + up
    xq = jax.nn.relu(_gcn_dense(A1, xq, Wu1, bu1))
    up = jnp.zeros((n, hid), jnp.float32).at[perm0].set(xq[:k0], mode="drop")
    xq = x1 + up
    xq = gcn0(xq, Wu2, bu2)
    return jax.nn.log_softmax(xq, axis=1)


# speedup vs baseline: 3.2171x; 1.0279x over previous
import math
import functools

import jax
import jax.numpy as jnp
from jax import lax
from jax.experimental import pallas as pl
from jax.experimental.pallas import tpu as pltpu
from jax.experimental.pallas import tpu_sc as plsc

_DEPTH = 3
_RATIOS = (2000.0 / 2708.0, 0.5, 0.5)
_NT = 16
_RG = 96
_RH = 128


def _sc_build_gh(src, dst, inv, perm, k0p, np_):
    E = src.shape[0]
    assert E % (_NT * 16) == 0 and k0p % _RG == 0 and np_ % _RH == 0
    assert _RG * np_ == _RH * k0p
    ept = E // _NT
    ncg = k0p // _RG
    nch = np_ // _RH
    dump = _RG * np_
    import functools as _ft

    mesh = plsc.VectorSubcoreMesh(core_axis_name="c", subcore_axis_name="s")

    @_ft.partial(
        pl.kernel,
        mesh=mesh,
        out_type=(
            jax.ShapeDtypeStruct((k0p * np_,), jnp.float32),
            jax.ShapeDtypeStruct((np_ * k0p,), jnp.float32),
        ),
        scratch_types=[
            pltpu.VMEM((ept,), jnp.int32),
            pltpu.VMEM((ept,), jnp.int32),
            pltpu.VMEM((np_,), jnp.int32),
            pltpu.VMEM((k0p,), jnp.int32),
            pltpu.VMEM((np_,), jnp.float32),
            pltpu.VMEM((16,), jnp.float32),
            pltpu.VMEM((ept + 16,), jnp.int32),
            pltpu.VMEM((ept,), jnp.int32),
            pltpu.VMEM_SHARED((_RG * np_ + 16,), jnp.float32),
        ],
        compiler_params=pltpu.CompilerParams(needs_layout_passes=False),
    )
    def built(src_h, dst_h, inv_h, perm_h, zeros_h, ones_h, gp_h, h0_h,
              src_v, dst_v, inv_v, perm_v, zeros_v, ones_v, list_v, flat_v,
              spm):
        core = lax.axis_index("c")
        tile = lax.axis_index("s")
        pltpu.sync_copy(src_h.at[pl.ds(tile * ept, ept)], src_v)
        pltpu.sync_copy(dst_h.at[pl.ds(tile * ept, ept)], dst_v)
        pltpu.sync_copy(inv_h, inv_v)
        pltpu.sync_copy(perm_h, perm_v)
        pltpu.sync_copy(zeros_h, zeros_v)
        pltpu.sync_copy(ones_h, ones_v)
        iota16 = jnp.arange(16, dtype=jnp.int32)

        def one_matrix(nchunks, rows, cols, out_ref, rowinv, diag):
            srows = rows // _NT

            def pre(g, cy):
                s16 = src_v[pl.ds(g * 16, 16)]
                d16 = dst_v[pl.ds(g * 16, 16)]
                if rowinv:
                    row = plsc.load_gather(inv_v, [s16])
                    col = d16
                    ok = (row >= 0) & (s16 != d16)
                else:
                    row = s16
                    col = plsc.load_gather(inv_v, [d16])
                    ok = (col >= 0) & (s16 != d16)
                flat_v[pl.ds(g * 16, 16)] = jnp.where(ok, row * cols + col,
                                                      -1)
                return cy
            lax.fori_loop(0, ept // 16, pre, 0)

            def chunk_body(c, carry):
                @pl.when(c % 2 == core)
                def _chunk():
                    row0 = c * rows

                    def zrow(r, cy):
                        base = (tile * srows + r) * cols
                        pltpu.sync_copy(zeros_v.at[pl.ds(0, cols)],
                                        spm.at[pl.ds(base, cols)])
                        return cy
                    lax.fori_loop(0, srows, zrow, 0)

                    @pl.when(tile == 0)
                    def _zdump():
                        pltpu.sync_copy(zeros_v.at[pl.ds(0, 16)],
                                        spm.at[pl.ds(_RG * np_, 16)])
                    plsc.subcore_barrier()

                    if diag:
                        @pl.when(tile < rows // 16)
                        def _diag():
                            a16 = row0 + tile * 16 + iota16
                            p16 = plsc.load_gather(perm_v, [a16])
                            fl = jnp.where(p16 >= 0,
                                           (a16 - row0) * cols + p16, dump)
                            pltpu.sync_copy(ones_v, spm.at[fl], add=True)
                            pltpu.sync_copy(ones_v, spm.at[fl], add=True)

                    lo = row0 * cols
                    hi = lo + rows * cols

                    def scan(g, cnt):
                        f16 = flat_v[pl.ds(g * 16, 16)]
                        m = (f16 >= lo) & (f16 < hi)
                        plsc.store_compressed(list_v.at[pl.ds(cnt, 16)],
                                              f16 - lo, mask=m)
                        return cnt + jnp.sum(m.astype(jnp.int32))
                    cnt = lax.fori_loop(0, ept // 16, scan, 0)
                    list_v[pl.ds(cnt, 16)] = jnp.full((16,), dump, jnp.int32)

                    def scat(j, cy):
                        idx16 = list_v[pl.ds(j * 16, 16)]
                        pltpu.sync_copy(ones_v, spm.at[idx16], add=True)
                        return cy
                    lax.fori_loop(0, (cnt + 15) // 16, scat, 0)
                    plsc.subcore_barrier()

                    sz = srows * cols
                    pltpu.sync_copy(
                        spm.at[pl.ds(tile * sz, sz)],
                        out_ref.at[pl.ds(row0 * cols + tile * sz, sz)])
                return carry
            lax.fori_loop(0, nchunks, chunk_body, 0)

        one_matrix(ncg, _RG, np_, gp_h, rowinv=True, diag=True)
        one_matrix(nch, _RH, k0p, h0_h, rowinv=False, diag=False)

    zeros = jnp.zeros((np_,), jnp.float32)
    ones = jnp.ones((16,), jnp.float32)
    gp, h0 = built(src, dst, inv, perm, zeros, ones)
    return gp.reshape(k0p, np_), h0.reshape(np_, k0p)


def _sc_deg(srcp, dstp, E, np_):
    EP = srcp.shape[0]
    ept = EP // 32
    assert ept % 16 == 0
    mesh = plsc.VectorSubcoreMesh(core_axis_name="c", subcore_axis_name="s")

    @functools.partial(
        pl.kernel,
        mesh=mesh,
        out_type=jax.ShapeDtypeStruct((2, 2 * np_), jnp.float32),
        scratch_types=[
            pltpu.VMEM((ept,), jnp.int32),
            pltpu.VMEM((ept,), jnp.int32),
            pltpu.VMEM((np_,), jnp.float32),
            pltpu.VMEM((16,), jnp.float32),
            pltpu.VMEM_SHARED((2 * np_ + 16,), jnp.float32),
        ],
        compiler_params=pltpu.CompilerParams(needs_layout_passes=False),
    )
    def built(src_h, dst_h, zeros_h, ones_h, out_h, src_v, dst_v, zeros_v,
              ones_v, spm):
        core = lax.axis_index("c")
        tile = lax.axis_index("s")
        base = (core * _NT + tile) * ept
        pltpu.sync_copy(src_h.at[pl.ds(base, ept)], src_v)
        pltpu.sync_copy(dst_h.at[pl.ds(base, ept)], dst_v)
        pltpu.sync_copy(zeros_h, zeros_v)
        pltpu.sync_copy(ones_h, ones_v)
        iota16 = jnp.arange(16, dtype=jnp.int32)
        dump = 2 * np_

        @pl.when(tile < 2)
        def _z():
            pltpu.sync_copy(zeros_v, spm.at[pl.ds(tile * np_, np_)])

        @pl.when(tile == 2)
        def _z2():
            pltpu.sync_copy(zeros_v.at[pl.ds(0, 16)], spm.at[pl.ds(dump, 16)])
        plsc.subcore_barrier()

        def scan(g, cy):
            s16 = src_v[pl.ds(g * 16, 16)]
            d16 = dst_v[pl.ds(g * 16, 16)]
            mv = base + g * 16 + iota16 < E
            pltpu.sync_copy(ones_v, spm.at[jnp.where(mv, s16, dump)], add=True)
            selfm = mv & (s16 == d16)
            pltpu.sync_copy(ones_v, spm.at[jnp.where(selfm, np_ + s16, dump)],
                            add=True)
            return cy
        lax.fori_loop(0, ept // 16, scan, 0)
        plsc.subcore_barrier()

        @pl.when(tile < 2)
        def _wb():
            pltpu.sync_copy(spm.at[pl.ds(tile * np_, np_)],
                            out_h.at[core].at[pl.ds(tile * np_, np_)])

    zeros = jnp.zeros((np_,), jnp.float32)
    ones = jnp.ones((16,), jnp.float32)
    out = built(srcp, dstp, zeros, ones)
    comb = out[0] + out[1]
    return comb[:np_], comb[np_:]


def _sc_msg_chunked(src, dst, u2, np_, nch, nr):
    E = src.shape[0]
    ept = E // _NT
    rchunk = np_ // nr
    assert ept % 16 == 0 and rchunk % _NT == 0 and (rchunk + 16) % _NT == 0
    zrows = (rchunk + 16) // _NT
    mesh = plsc.VectorSubcoreMesh(core_axis_name="c", subcore_axis_name="s")

    @functools.partial(
        pl.kernel,
        mesh=mesh,
        out_type=jax.ShapeDtypeStruct((nch * np_, 128), jnp.float32),
        scratch_types=[
            pltpu.VMEM((ept,), jnp.int32),
            pltpu.VMEM((ept,), jnp.int32),
            pltpu.VMEM((zrows, 128), jnp.float32),
            pltpu.VMEM((16, 128), jnp.float32),
            pltpu.SemaphoreType.DMA,
            pltpu.VMEM_SHARED((rchunk + 16, 128), jnp.float32),
        ],
        compiler_params=pltpu.CompilerParams(needs_layout_passes=False),
    )
    def built(src_h, dst_h, zeros_h, u_h, out_h, src_v, dst_v, zeros_v,
              rows_v, sem, spm):
        core = lax.axis_index("c")
        tile = lax.axis_index("s")
        pltpu.sync_copy(src_h.at[pl.ds(tile * ept, ept)], src_v)
        pltpu.sync_copy(dst_h.at[pl.ds(tile * ept, ept)], dst_v)
        pltpu.sync_copy(zeros_h, zeros_v)

        def chunk(ci, carry):
            @pl.when(ci % 2 == core)
            def _chunk():
                cch = ci // nr
                r0 = (ci % nr) * rchunk
                pltpu.sync_copy(zeros_v, spm.at[pl.ds(tile * zrows, zrows)])
                plsc.subcore_barrier()

                def scan(g, cy):
                    s16 = src_v[pl.ds(g * 16, 16)]
                    d16 = dst_v[pl.ds(g * 16, 16)]
                    ok = ((s16 != d16) & (s16 >= r0)
                          & (s16 < r0 + rchunk))

                    @pl.when(jnp.any(ok))
                    def _go():
                        pltpu.async_copy(
                            u_h.at[cch * np_ + jnp.where(ok, d16, 0)],
                            rows_v, sem).wait()
                        pltpu.sync_copy(
                            rows_v,
                            spm.at[jnp.where(ok, s16 - r0, rchunk)],
                            add=True)
                    return cy
                lax.fori_loop(0, ept // 16, scan, 0)
                plsc.subcore_barrier()

                wb = rchunk // _NT
                pltpu.sync_copy(
                    spm.at[pl.ds(tile * wb, wb)],
                    out_h.at[pl.ds(cch * np_ + r0 + tile * wb, wb)])
            return carry
        lax.fori_loop(0, nch * nr, chunk, 0)

    zeros = jnp.zeros((zrows, 128), jnp.float32)
    return built(src, dst, zeros, u2)


def _ceil_to(v, m):
    return ((v + m - 1) // m) * m


def _pick(dim, cands):
    for c in cands:
        if dim % c == 0:
            return c
    raise AssertionError(dim)


def _mm(a, b, *, zero_diag=False, out_dtype=jnp.float32, split_b=False,
        bn_cap=4096):
    M, K = a.shape
    K2, N = b.shape
    assert K == K2, (a.shape, b.shape)
    bm = _pick(M, (1536, 768, 512, 256, 128))
    bn = _pick(N, (1536, 768, 512, 256, 128))
    bn = min(bn, bn_cap)
    bk = _pick(K, (512, 768, 256, 128))
    nk = K // bk

    def kern(a_ref, b_ref, o_ref, acc_ref):
        @pl.when(pl.program_id(2) == 0)
        def _():
            acc_ref[...] = jnp.zeros_like(acc_ref)

        av = a_ref[...].astype(jnp.bfloat16)
        bv = b_ref[...]
        if split_b:
            bhi = bv.astype(jnp.bfloat16)
            blo = (bv - bhi.astype(jnp.float32)).astype(jnp.bfloat16)
            acc_ref[...] += (jnp.dot(av, bhi, preferred_element_type=jnp.float32)
                             + jnp.dot(av, blo, preferred_element_type=jnp.float32))
        else:
            acc_ref[...] += jnp.dot(av, bv.astype(jnp.bfloat16),
                                    preferred_element_type=jnp.float32)

        if zero_diag:
            gi = pl.program_id(0) * bm + lax.broadcasted_iota(jnp.int32, (bm, bn), 0)
            gj = pl.program_id(1) * bn + lax.broadcasted_iota(jnp.int32, (bm, bn), 1)
            diag = gi == gj
        else:
            diag = None
        last = pl.program_id(2) == nk - 1

        @pl.when(last)
        def _():
            r = acc_ref[...]
            if diag is not None:
                r = jnp.where(diag, 0.0, r)
            o_ref[...] = r.astype(out_dtype)

    return pl.pallas_call(
        kern,
        grid=(M // bm, N // bn, nk),
        in_specs=[
            pl.BlockSpec((bm, bk), lambda i, j, k: (i, k)),
            pl.BlockSpec((bk, bn), lambda i, j, k: (k, j)),
        ],
        out_specs=pl.BlockSpec((bm, bn), lambda i, j, k: (i, j)),
        out_shape=jax.ShapeDtypeStruct((M, N), out_dtype),
        scratch_shapes=[pltpu.VMEM((bm, bn), jnp.float32)],
        compiler_params=pltpu.CompilerParams(
            dimension_semantics=("parallel", "parallel", "arbitrary")),
    )(a, b)


def _mm_f32(a, b):
    M, K = a.shape
    _, N = b.shape
    bm = _pick(M, (512, 256, 128))
    bn = _pick(N, (256, 128))
    bk = _pick(K, (512, 256, 128, 32))
    nk = K // bk

    def kern(a_ref, b_ref, o_ref, acc_ref):
        @pl.when(pl.program_id(2) == 0)
        def _():
            acc_ref[...] = jnp.zeros_like(acc_ref)

        acc_ref[...] += jnp.dot(a_ref[...], b_ref[...],
                                preferred_element_type=jnp.float32,
                                precision=lax.Precision.HIGHEST)
        last = pl.program_id(2) == nk - 1

        @pl.when(last)
        def _():
            o_ref[...] = acc_ref[...]

    return pl.pallas_call(
        kern,
        grid=(M // bm, N // bn, nk),
        in_specs=[
            pl.BlockSpec((bm, bk), lambda i, j, k: (i, k)),
            pl.BlockSpec((bk, bn), lambda i, j, k: (k, j)),
        ],
        out_specs=pl.BlockSpec((bm, bn), lambda i, j, k: (i, j)),
        out_shape=jax.ShapeDtypeStruct((M, N), jnp.float32),
        scratch_shapes=[pltpu.VMEM((bm, bn), jnp.float32)],
        compiler_params=pltpu.CompilerParams(
            dimension_semantics=("parallel", "parallel", "arbitrary")),
    )(a, b)


def _dis_from_rowsum(Al):
    M, N = Al.shape
    bm = _pick(M, (768, 512, 256, 128))
    bn = _pick(N, (768, 512, 256, 128))
    nj = N // bn

    def kern(a_ref, o_ref, acc_ref):
        @pl.when(pl.program_id(1) == 0)
        def _():
            acc_ref[...] = jnp.zeros_like(acc_ref)

        acc_ref[...] += jnp.sum(a_ref[...].astype(jnp.float32), axis=1,
                                keepdims=True)

        @pl.when(pl.program_id(1) == nj - 1)
        def _():
            o_ref[...] = lax.rsqrt(acc_ref[...] + 2.0)

    return pl.pallas_call(
        kern,
        grid=(M // bm, nj),
        in_specs=[pl.BlockSpec((bm, bn), lambda i, j: (i, j))],
        out_specs=pl.BlockSpec((bm, 1), lambda i, j: (i, 0)),
        out_shape=jax.ShapeDtypeStruct((M, 1), jnp.float32),
        scratch_shapes=[pltpu.VMEM((bm, 1), jnp.float32)],
        compiler_params=pltpu.CompilerParams(
            dimension_semantics=("parallel", "arbitrary")),
    )(Al)


def _pad2(a, m, n):
    return jnp.pad(a, ((0, m - a.shape[0]), (0, n - a.shape[1])))


def _mm_feat(a, w):
    M = a.shape[0]
    K = _ceil_to(a.shape[1], 32)
    N = w.shape[1]
    Np = _ceil_to(N, 128)
    ap = _pad2(a, M, K)
    wp = _pad2(w, K, Np)
    r = _mm_f32(ap, wp)
    return r[:, :N]


def _gcn_dense(Al, xl, W, b):
    dis = _dis_from_rowsum(Al)
    xw = _mm_feat(xl, W)
    u = dis * xw
    kp = Al.shape[0]
    up = _pad2(u, kp, 128)
    m = _mm(Al, up, split_b=True)[:, : xw.shape[1]]
    out = dis * m + (2.0 * dis * dis) * xw + b[None, :]
    return out


_RATIOS_K = {}


def _pool(xl, w, nreal):
    nl = xl.shape[0]
    k = int(math.ceil(_RATIOS_K[nreal]))
    score = jnp.tanh((xl @ w) / jnp.linalg.norm(w))
    score = jnp.where(jnp.arange(nl) < nreal, score, -2.0)
    _, permd = lax.top_k(score, k)
    perm = jnp.sort(permd)
    sv = score[perm]
    xp = xl[perm] * sv[:, None]
    inv = jnp.full((nl,), -1, jnp.int32).at[perm].set(jnp.arange(k, dtype=jnp.int32))
    return perm, inv, xp, k


def kernel(x, edge_index, Wd0, bd0, Wd1, bd1, Wd2, bd2, Wd3, bd3,
           p0, p1, p2, Wu0, bu0, Wu1, bu1, Wu2, bu2):
    n = x.shape[0]
    hid = Wd0.shape[1]
    src, dst = edge_index[0], edge_index[1]
    selfm = src == dst

    k0 = int(math.ceil(_RATIOS[0] * n))
    k1 = int(math.ceil(_RATIOS[1] * k0))
    k2 = int(math.ceil(_RATIOS[2] * k1))
    np_ = _ceil_to(n, 512)
    k0p = _ceil_to(k0, 768 if k0 > 768 else 256)
    k1p = _ceil_to(k1, 768 if k1 > 768 else 256)
    k2p = _ceil_to(k2, 768 if k2 > 768 else 256)
    _RATIOS_K[n] = _RATIOS[0] * n
    _RATIOS_K[k0] = _RATIOS[1] * k0
    _RATIOS_K[k1] = _RATIOS[2] * k1

    E = src.shape[0]
    big = n >= 1024
    if big:
        EP2 = _ceil_to(E, 512)
        srcp = jnp.pad(src, (0, EP2 - E))
        dstp = jnp.pad(dst, (0, EP2 - E))
        rowsum0, c0 = _sc_deg(srcp, dstp, E, np_)
        rowsum0, c0 = rowsum0[:n], c0[:n]
    else:
        ones_e = jnp.ones(src.shape, jnp.float32)
        rowsum0 = jnp.zeros((n,), jnp.float32).at[src].add(ones_e)
        c0 = jnp.zeros((n,), jnp.float32).at[src].add(
            jnp.where(selfm, 1.0, 0.0))
    a2ii = jnp.where(c0 > 0, c0, 2.0)
    deg0 = rowsum0 - c0 + a2ii
    dis0 = lax.rsqrt(deg0)

    def msgpass(u):
        return jnp.zeros_like(u).at[src].add(
            jnp.where(selfm, 0.0, 1.0)[:, None] * u[dst])

    def gcn0(xl, W, b):
        xw = _mm_feat(_pad2(xl, np_, xl.shape[1]), W)[:n]
        if xl.shape[1] <= W.shape[1]:
            m32 = msgpass(dis0[:, None] * xl)
            msg = _mm_feat(_pad2(m32, np_, m32.shape[1]), W)[:n]
        else:
            msg = msgpass(dis0[:, None] * xw)
        return dis0[:, None] * msg + (a2ii * dis0 * dis0)[:, None] * xw + b[None, :]

    x1 = jax.nn.relu(gcn0(x, Wd0, bd0))

    x1p = _pad2(x1, np_, hid)
    perm0, inv0, xp0, _ = _pool(x1p, p0, n)
    if n >= 1024:
        perm0p = jnp.pad(perm0, (0, k0p - k0), constant_values=-1)
        Gp, H0 = _sc_build_gh(src, dst, inv0, perm0p, k0p, np_)
    else:
        one = jnp.ones((), jnp.float32)
        gi = jnp.where((inv0[src] >= 0) & (~selfm), inv0[src], k0p)
        Gp = jnp.zeros((k0p, np_), jnp.float32).at[gi, dst].add(one, mode="drop")
        Gp = Gp.at[jnp.arange(k0), perm0[:k0]].add(2.0, mode="drop")
        hi = jnp.where((inv0[dst] >= 0) & (~selfm), inv0[dst], k0p)
        H0 = jnp.zeros((np_, k0p), jnp.float32).at[src, hi].add(one, mode="drop")
    A1 = _mm(Gp, H0, zero_diag=True, out_dtype=jnp.bfloat16)

    xp0 = _pad2(xp0[:k0], k0p, hid)
    x2 = jax.nn.relu(_gcn_dense(A1, xp0, Wd1, bd1))

    perm1, inv1, xp1, _ = _pool(x2, p1, k0)
    G1 = jnp.pad(A1[perm1[:k1]], ((0, k1p - k1), (0, 0)))
    G1 = G1.at[jnp.arange(k1), perm1[:k1]].add(jnp.bfloat16(2.0))
    H1 = jnp.pad(A1[:, perm1[:k1]], ((0, 0), (0, k1p - k1)))
    A2 = _mm(G1, H1, zero_diag=True, out_dtype=jnp.bfloat16)

    xp1 = _pad2(xp1[:k1], k1p, hid)
    x3 = jax.nn.relu(_gcn_dense(A2, xp1, Wd2, bd2))

    perm2, inv2, xp2, _ = _pool(x3, p2, k1)
    G2 = jnp.pad(A2[perm2[:k2]], ((0, k2p - k2), (0, 0)))
    G2 = G2.at[jnp.arange(k2), perm2[:k2]].add(jnp.bfloat16(2.0))
    H2 = jnp.pad(A2[:, perm2[:k2]], ((0, 0), (0, k2p - k2)))
    A3 = _mm(G2, H2, zero_diag=True, out_dtype=jnp.bfloat16)

    xp2 = _pad2(xp2[:k2], k2p, hid)
    x4 = jax.nn.relu(_gcn_dense(A3, xp2, Wd3, bd3))

    up = jnp.zeros_like(x3).at[perm2[:k2]].set(x4[:k2])
    xq = x3 + up
    xq = jax.nn.relu(_gcn_dense(A2, xq, Wu0, bu0))
    up = jnp.zeros_like(x2).at[perm1[:k1]].set(xq[:k1])
    xq = x2 + up
    xq = jax.nn.relu(_gcn_dense(A1, xq, Wu1, bu1))
    up = jnp.zeros((n, hid), jnp.float32).at[perm0].set(xq[:k0], mode="drop")
    xq = x1 + up
    xq = gcn0(xq, Wu2, bu2)
    return jax.nn.log_softmax(xq, axis=1)


# speedup vs baseline: 3.3072x; 1.0280x over previous
import math
import functools

import jax
import jax.numpy as jnp
from jax import lax
from jax.experimental import pallas as pl
from jax.experimental.pallas import tpu as pltpu
from jax.experimental.pallas import tpu_sc as plsc

_DEPTH = 3
_RATIOS = (2000.0 / 2708.0, 0.5, 0.5)
_NT = 16
_RG = 96
_RH = 128


def _sc_build_gh(src, dst, inv, perm, k0p, np_):
    E = src.shape[0]
    assert E % (_NT * 16) == 0 and k0p % _RG == 0 and np_ % _RH == 0
    assert _RG * np_ == _RH * k0p
    ept = E // _NT
    ncg = k0p // _RG
    nch = np_ // _RH
    dump = _RG * np_
    import functools as _ft

    mesh = plsc.VectorSubcoreMesh(core_axis_name="c", subcore_axis_name="s")

    @_ft.partial(
        pl.kernel,
        mesh=mesh,
        out_type=(
            jax.ShapeDtypeStruct((k0p * np_,), jnp.float32),
            jax.ShapeDtypeStruct((np_ * k0p,), jnp.float32),
        ),
        scratch_types=[
            pltpu.VMEM((ept,), jnp.int32),
            pltpu.VMEM((ept,), jnp.int32),
            pltpu.VMEM((np_,), jnp.int32),
            pltpu.VMEM((k0p,), jnp.int32),
            pltpu.VMEM((np_,), jnp.float32),
            pltpu.VMEM((16,), jnp.float32),
            pltpu.VMEM((ept + 16,), jnp.int32),
            pltpu.VMEM((ept,), jnp.int32),
            pltpu.VMEM_SHARED((_RG * np_ + 16,), jnp.float32),
        ],
        compiler_params=pltpu.CompilerParams(needs_layout_passes=False),
    )
    def built(src_h, dst_h, inv_h, perm_h, zeros_h, ones_h, gp_h, h0_h,
              src_v, dst_v, inv_v, perm_v, zeros_v, ones_v, list_v, flat_v,
              spm):
        core = lax.axis_index("c")
        tile = lax.axis_index("s")
        pltpu.sync_copy(src_h.at[pl.ds(tile * ept, ept)], src_v)
        pltpu.sync_copy(dst_h.at[pl.ds(tile * ept, ept)], dst_v)
        pltpu.sync_copy(inv_h, inv_v)
        pltpu.sync_copy(perm_h, perm_v)
        pltpu.sync_copy(zeros_h, zeros_v)
        pltpu.sync_copy(ones_h, ones_v)
        iota16 = jnp.arange(16, dtype=jnp.int32)

        def one_matrix(nchunks, rows, cols, out_ref, rowinv, diag):
            srows = rows // _NT

            def pre(g, cy):
                s16 = src_v[pl.ds(g * 16, 16)]
                d16 = dst_v[pl.ds(g * 16, 16)]
                if rowinv:
                    row = plsc.load_gather(inv_v, [s16])
                    col = d16
                    ok = (row >= 0) & (s16 != d16)
                else:
                    row = s16
                    col = plsc.load_gather(inv_v, [d16])
                    ok = (col >= 0) & (s16 != d16)
                flat_v[pl.ds(g * 16, 16)] = jnp.where(ok, row * cols + col,
                                                      -1)
                return cy
            lax.fori_loop(0, ept // 16, pre, 0)

            def chunk_body(c, carry):
                @pl.when(c % 2 == core)
                def _chunk():
                    row0 = c * rows

                    def zrow(r, cy):
                        base = (tile * srows + r) * cols
                        pltpu.sync_copy(zeros_v.at[pl.ds(0, cols)],
                                        spm.at[pl.ds(base, cols)])
                        return cy
                    lax.fori_loop(0, srows, zrow, 0)

                    @pl.when(tile == 0)
                    def _zdump():
                        pltpu.sync_copy(zeros_v.at[pl.ds(0, 16)],
                                        spm.at[pl.ds(_RG * np_, 16)])
                    plsc.subcore_barrier()

                    if diag:
                        @pl.when(tile < rows // 16)
                        def _diag():
                            a16 = row0 + tile * 16 + iota16
                            p16 = plsc.load_gather(perm_v, [a16])
                            fl = jnp.where(p16 >= 0,
                                           (a16 - row0) * cols + p16, dump)
                            pltpu.sync_copy(ones_v, spm.at[fl], add=True)
                            pltpu.sync_copy(ones_v, spm.at[fl], add=True)

                    lo = row0 * cols
                    hi = lo + rows * cols

                    def scan(g, cnt):
                        f16 = flat_v[pl.ds(g * 16, 16)]
                        m = (f16 >= lo) & (f16 < hi)
                        plsc.store_compressed(list_v.at[pl.ds(cnt, 16)],
                                              f16 - lo, mask=m)
                        return cnt + jnp.sum(m.astype(jnp.int32))
                    cnt = lax.fori_loop(0, ept // 16, scan, 0)
                    list_v[pl.ds(cnt, 16)] = jnp.full((16,), dump, jnp.int32)

                    def scat(j, cy):
                        idx16 = list_v[pl.ds(j * 16, 16)]
                        pltpu.sync_copy(ones_v, spm.at[idx16], add=True)
                        return cy
                    lax.fori_loop(0, (cnt + 15) // 16, scat, 0)
                    plsc.subcore_barrier()

                    sz = srows * cols
                    pltpu.sync_copy(
                        spm.at[pl.ds(tile * sz, sz)],
                        out_ref.at[pl.ds(row0 * cols + tile * sz, sz)])
                return carry
            lax.fori_loop(0, nchunks, chunk_body, 0)

        one_matrix(ncg, _RG, np_, gp_h, rowinv=True, diag=True)
        one_matrix(nch, _RH, k0p, h0_h, rowinv=False, diag=False)

    zeros = jnp.zeros((np_,), jnp.float32)
    ones = jnp.ones((16,), jnp.float32)
    gp, h0 = built(src, dst, inv, perm, zeros, ones)
    return gp.reshape(k0p, np_), h0.reshape(np_, k0p)


def _sc_deg(srcp, dstp, E, np_):
    EP = srcp.shape[0]
    ept = EP // 32
    assert ept % 16 == 0
    mesh = plsc.VectorSubcoreMesh(core_axis_name="c", subcore_axis_name="s")

    @functools.partial(
        pl.kernel,
        mesh=mesh,
        out_type=jax.ShapeDtypeStruct((2, 2 * np_), jnp.float32),
        scratch_types=[
            pltpu.VMEM((ept,), jnp.int32),
            pltpu.VMEM((ept,), jnp.int32),
            pltpu.VMEM((np_,), jnp.float32),
            pltpu.VMEM((16,), jnp.float32),
            pltpu.VMEM_SHARED((2 * np_ + 16,), jnp.float32),
        ],
        compiler_params=pltpu.CompilerParams(needs_layout_passes=False),
    )
    def built(src_h, dst_h, zeros_h, ones_h, out_h, src_v, dst_v, zeros_v,
              ones_v, spm):
        core = lax.axis_index("c")
        tile = lax.axis_index("s")
        base = (core * _NT + tile) * ept
        pltpu.sync_copy(src_h.at[pl.ds(base, ept)], src_v)
        pltpu.sync_copy(dst_h.at[pl.ds(base, ept)], dst_v)
        pltpu.sync_copy(zeros_h, zeros_v)
        pltpu.sync_copy(ones_h, ones_v)
        iota16 = jnp.arange(16, dtype=jnp.int32)
        dump = 2 * np_

        @pl.when(tile < 2)
        def _z():
            pltpu.sync_copy(zeros_v, spm.at[pl.ds(tile * np_, np_)])

        @pl.when(tile == 2)
        def _z2():
            pltpu.sync_copy(zeros_v.at[pl.ds(0, 16)], spm.at[pl.ds(dump, 16)])
        plsc.subcore_barrier()

        def scan(g, cy):
            s16 = src_v[pl.ds(g * 16, 16)]
            d16 = dst_v[pl.ds(g * 16, 16)]
            mv = base + g * 16 + iota16 < E
            pltpu.sync_copy(ones_v, spm.at[jnp.where(mv, s16, dump)], add=True)
            selfm = mv & (s16 == d16)
            pltpu.sync_copy(ones_v, spm.at[jnp.where(selfm, np_ + s16, dump)],
                            add=True)
            return cy
        lax.fori_loop(0, ept // 16, scan, 0)
        plsc.subcore_barrier()

        @pl.when(tile < 2)
        def _wb():
            pltpu.sync_copy(spm.at[pl.ds(tile * np_, np_)],
                            out_h.at[core].at[pl.ds(tile * np_, np_)])

    zeros = jnp.zeros((np_,), jnp.float32)
    ones = jnp.ones((16,), jnp.float32)
    out = built(srcp, dstp, zeros, ones)
    comb = out[0] + out[1]
    return comb[:np_], comb[np_:]


def _sc_msg_chunked(src, dst, u2, np_, nch, nr):
    E = src.shape[0]
    ept = E // _NT
    rchunk = np_ // nr
    assert ept % 16 == 0 and rchunk % _NT == 0 and (rchunk + 16) % _NT == 0
    zrows = (rchunk + 16) // _NT
    mesh = plsc.VectorSubcoreMesh(core_axis_name="c", subcore_axis_name="s")

    @functools.partial(
        pl.kernel,
        mesh=mesh,
        out_type=jax.ShapeDtypeStruct((nch * np_, 128), jnp.float32),
        scratch_types=[
            pltpu.VMEM((ept,), jnp.int32),
            pltpu.VMEM((ept,), jnp.int32),
            pltpu.VMEM((zrows, 128), jnp.float32),
            pltpu.VMEM((16, 128), jnp.float32),
            pltpu.SemaphoreType.DMA,
            pltpu.VMEM_SHARED((rchunk + 16, 128), jnp.float32),
        ],
        compiler_params=pltpu.CompilerParams(needs_layout_passes=False),
    )
    def built(src_h, dst_h, zeros_h, u_h, out_h, src_v, dst_v, zeros_v,
              rows_v, sem, spm):
        core = lax.axis_index("c")
        tile = lax.axis_index("s")
        pltpu.sync_copy(src_h.at[pl.ds(tile * ept, ept)], src_v)
        pltpu.sync_copy(dst_h.at[pl.ds(tile * ept, ept)], dst_v)
        pltpu.sync_copy(zeros_h, zeros_v)

        def chunk(ci, carry):
            @pl.when(ci % 2 == core)
            def _chunk():
                cch = ci // nr
                r0 = (ci % nr) * rchunk
                pltpu.sync_copy(zeros_v, spm.at[pl.ds(tile * zrows, zrows)])
                plsc.subcore_barrier()

                def scan(g, cy):
                    s16 = src_v[pl.ds(g * 16, 16)]
                    d16 = dst_v[pl.ds(g * 16, 16)]
                    ok = ((s16 != d16) & (s16 >= r0)
                          & (s16 < r0 + rchunk))

                    @pl.when(jnp.any(ok))
                    def _go():
                        pltpu.async_copy(
                            u_h.at[cch * np_ + jnp.where(ok, d16, 0)],
                            rows_v, sem).wait()
                        pltpu.sync_copy(
                            rows_v,
                            spm.at[jnp.where(ok, s16 - r0, rchunk)],
                            add=True)
                    return cy
                lax.fori_loop(0, ept // 16, scan, 0)
                plsc.subcore_barrier()

                wb = rchunk // _NT
                pltpu.sync_copy(
                    spm.at[pl.ds(tile * wb, wb)],
                    out_h.at[pl.ds(cch * np_ + r0 + tile * wb, wb)])
            return carry
        lax.fori_loop(0, nch * nr, chunk, 0)

    zeros = jnp.zeros((zrows, 128), jnp.float32)
    return built(src, dst, zeros, u2)


def _ceil_to(v, m):
    return ((v + m - 1) // m) * m


def _pick(dim, cands):
    for c in cands:
        if dim % c == 0:
            return c
    raise AssertionError(dim)


def _mm(a, b, *, zero_diag=False, out_dtype=jnp.float32, split_b=False,
        bn_cap=4096):
    M, K = a.shape
    K2, N = b.shape
    assert K == K2, (a.shape, b.shape)
    bm = _pick(M, (1920, 1536, 768, 512, 256, 128))
    bn = _pick(N, (1920, 1536, 768, 512, 256, 128))
    bn = min(bn, bn_cap)
    bk = _pick(K, (512, 768, 256, 128))
    nk = K // bk

    def kern(a_ref, b_ref, o_ref, acc_ref):
        @pl.when(pl.program_id(2) == 0)
        def _():
            acc_ref[...] = jnp.zeros_like(acc_ref)

        av = a_ref[...].astype(jnp.bfloat16)
        bv = b_ref[...]
        if split_b:
            bhi = bv.astype(jnp.bfloat16)
            blo = (bv - bhi.astype(jnp.float32)).astype(jnp.bfloat16)
            acc_ref[...] += (jnp.dot(av, bhi, preferred_element_type=jnp.float32)
                             + jnp.dot(av, blo, preferred_element_type=jnp.float32))
        else:
            acc_ref[...] += jnp.dot(av, bv.astype(jnp.bfloat16),
                                    preferred_element_type=jnp.float32)

        if zero_diag:
            gi = pl.program_id(0) * bm + lax.broadcasted_iota(jnp.int32, (bm, bn), 0)
            gj = pl.program_id(1) * bn + lax.broadcasted_iota(jnp.int32, (bm, bn), 1)
            diag = gi == gj
        else:
            diag = None
        last = pl.program_id(2) == nk - 1

        @pl.when(last)
        def _():
            r = acc_ref[...]
            if diag is not None:
                r = jnp.where(diag, 0.0, r)
            o_ref[...] = r.astype(out_dtype)

    return pl.pallas_call(
        kern,
        grid=(M // bm, N // bn, nk),
        in_specs=[
            pl.BlockSpec((bm, bk), lambda i, j, k: (i, k)),
            pl.BlockSpec((bk, bn), lambda i, j, k: (k, j)),
        ],
        out_specs=pl.BlockSpec((bm, bn), lambda i, j, k: (i, j)),
        out_shape=jax.ShapeDtypeStruct((M, N), out_dtype),
        scratch_shapes=[pltpu.VMEM((bm, bn), jnp.float32)],
        compiler_params=pltpu.CompilerParams(
            dimension_semantics=("parallel", "parallel", "arbitrary")),
    )(a, b)


def _mm_f32(a, b):
    M, K = a.shape
    _, N = b.shape
    bm = _pick(M, (512, 256, 128))
    bn = _pick(N, (256, 128))
    bk = _pick(K, (512, 256, 128, 32))
    nk = K // bk

    def kern(a_ref, b_ref, o_ref, acc_ref):
        @pl.when(pl.program_id(2) == 0)
        def _():
            acc_ref[...] = jnp.zeros_like(acc_ref)

        acc_ref[...] += jnp.dot(a_ref[...], b_ref[...],
                                preferred_element_type=jnp.float32,
                                precision=lax.Precision.HIGHEST)
        last = pl.program_id(2) == nk - 1

        @pl.when(last)
        def _():
            o_ref[...] = acc_ref[...]

    return pl.pallas_call(
        kern,
        grid=(M // bm, N // bn, nk),
        in_specs=[
            pl.BlockSpec((bm, bk), lambda i, j, k: (i, k)),
            pl.BlockSpec((bk, bn), lambda i, j, k: (k, j)),
        ],
        out_specs=pl.BlockSpec((bm, bn), lambda i, j, k: (i, j)),
        out_shape=jax.ShapeDtypeStruct((M, N), jnp.float32),
        scratch_shapes=[pltpu.VMEM((bm, bn), jnp.float32)],
        compiler_params=pltpu.CompilerParams(
            dimension_semantics=("parallel", "parallel", "arbitrary")),
    )(a, b)


def _dis_from_rowsum(Al):
    M, N = Al.shape
    bm = _pick(M, (768, 512, 256, 128))
    bn = _pick(N, (768, 512, 256, 128))
    nj = N // bn

    def kern(a_ref, o_ref, acc_ref):
        @pl.when(pl.program_id(1) == 0)
        def _():
            acc_ref[...] = jnp.zeros_like(acc_ref)

        acc_ref[...] += jnp.sum(a_ref[...].astype(jnp.float32), axis=1,
                                keepdims=True)

        @pl.when(pl.program_id(1) == nj - 1)
        def _():
            o_ref[...] = lax.rsqrt(acc_ref[...] + 2.0)

    return pl.pallas_call(
        kern,
        grid=(M // bm, nj),
        in_specs=[pl.BlockSpec((bm, bn), lambda i, j: (i, j))],
        out_specs=pl.BlockSpec((bm, 1), lambda i, j: (i, 0)),
        out_shape=jax.ShapeDtypeStruct((M, 1), jnp.float32),
        scratch_shapes=[pltpu.VMEM((bm, 1), jnp.float32)],
        compiler_params=pltpu.CompilerParams(
            dimension_semantics=("parallel", "arbitrary")),
    )(Al)


def _pad2(a, m, n):
    return jnp.pad(a, ((0, m - a.shape[0]), (0, n - a.shape[1])))


def _mm_feat(a, w):
    M = a.shape[0]
    K = _ceil_to(a.shape[1], 32)
    N = w.shape[1]
    Np = _ceil_to(N, 128)
    ap = _pad2(a, M, K)
    wp = _pad2(w, K, Np)
    r = _mm_f32(ap, wp)
    return r[:, :N]


def _gcn_dense(Al, xl, W, b):
    dis = _dis_from_rowsum(Al)
    xw = _mm_feat(xl, W)
    u = dis * xw
    kp = Al.shape[0]
    up = _pad2(u, kp, 128)
    m = _mm(Al, up, split_b=True)[:, : xw.shape[1]]
    out = dis * m + (2.0 * dis * dis) * xw + b[None, :]
    return out


_RATIOS_K = {}


def _pool(xl, w, nreal):
    nl = xl.shape[0]
    k = int(math.ceil(_RATIOS_K[nreal]))
    score = jnp.tanh((xl @ w) / jnp.linalg.norm(w))
    score = jnp.where(jnp.arange(nl) < nreal, score, -2.0)
    _, permd = lax.top_k(score, k)
    perm = jnp.sort(permd)
    sv = score[perm]
    xp = xl[perm] * sv[:, None]
    inv = jnp.full((nl,), -1, jnp.int32).at[perm].set(jnp.arange(k, dtype=jnp.int32))
    return perm, inv, xp, k


def kernel(x, edge_index, Wd0, bd0, Wd1, bd1, Wd2, bd2, Wd3, bd3,
           p0, p1, p2, Wu0, bu0, Wu1, bu1, Wu2, bu2):
    n = x.shape[0]
    hid = Wd0.shape[1]
    src, dst = edge_index[0], edge_index[1]
    selfm = src == dst

    k0 = int(math.ceil(_RATIOS[0] * n))
    k1 = int(math.ceil(_RATIOS[1] * k0))
    k2 = int(math.ceil(_RATIOS[2] * k1))
    np_ = _ceil_to(n, 512)
    k0p = _ceil_to(k0, 768 if k0 > 768 else 256)
    k1p = _ceil_to(k1, 768 if k1 > 768 else 256)
    k2p = _ceil_to(k2, 768 if k2 > 768 else 256)
    _RATIOS_K[n] = _RATIOS[0] * n
    _RATIOS_K[k0] = _RATIOS[1] * k0
    _RATIOS_K[k1] = _RATIOS[2] * k1

    E = src.shape[0]
    big = n >= 1024
    if big:
        EP2 = _ceil_to(E, 512)
        srcp = jnp.pad(src, (0, EP2 - E))
        dstp = jnp.pad(dst, (0, EP2 - E))
        rowsum0, c0 = _sc_deg(srcp, dstp, E, np_)
        rowsum0, c0 = rowsum0[:n], c0[:n]
    else:
        ones_e = jnp.ones(src.shape, jnp.float32)
        rowsum0 = jnp.zeros((n,), jnp.float32).at[src].add(ones_e)
        c0 = jnp.zeros((n,), jnp.float32).at[src].add(
            jnp.where(selfm, 1.0, 0.0))
    a2ii = jnp.where(c0 > 0, c0, 2.0)
    deg0 = rowsum0 - c0 + a2ii
    dis0 = lax.rsqrt(deg0)

    def msgpass(u):
        return jnp.zeros_like(u).at[src].add(
            jnp.where(selfm, 0.0, 1.0)[:, None] * u[dst])

    def gcn0(xl, W, b):
        xw = _mm_feat(_pad2(xl, np_, xl.shape[1]), W)[:n]
        if xl.shape[1] <= W.shape[1]:
            m32 = msgpass(dis0[:, None] * xl)
            msg = _mm_feat(_pad2(m32, np_, m32.shape[1]), W)[:n]
        else:
            msg = msgpass(dis0[:, None] * xw)
        return dis0[:, None] * msg + (a2ii * dis0 * dis0)[:, None] * xw + b[None, :]

    x1 = jax.nn.relu(gcn0(x, Wd0, bd0))

    x1p = _pad2(x1, np_, hid)
    perm0, inv0, xp0, _ = _pool(x1p, p0, n)
    if n >= 1024:
        perm0p = jnp.pad(perm0, (0, k0p - k0), constant_values=-1)
        Gp, H0 = _sc_build_gh(src, dst, inv0, perm0p, k0p, np_)
    else:
        one = jnp.ones((), jnp.float32)
        gi = jnp.where((inv0[src] >= 0) & (~selfm), inv0[src], k0p)
        Gp = jnp.zeros((k0p, np_), jnp.float32).at[gi, dst].add(one, mode="drop")
        Gp = Gp.at[jnp.arange(k0), perm0[:k0]].add(2.0, mode="drop")
        hi = jnp.where((inv0[dst] >= 0) & (~selfm), inv0[dst], k0p)
        H0 = jnp.zeros((np_, k0p), jnp.float32).at[src, hi].add(one, mode="drop")
    A1 = _mm(Gp, H0, zero_diag=True, out_dtype=jnp.bfloat16)

    xp0 = _pad2(xp0[:k0], k0p, hid)
    x2 = jax.nn.relu(_gcn_dense(A1, xp0, Wd1, bd1))

    perm1, inv1, xp1, _ = _pool(x2, p1, k0)
    G1 = jnp.pad(A1[perm1[:k1]], ((0, k1p - k1), (0, 0)))
    G1 = G1.at[jnp.arange(k1), perm1[:k1]].add(jnp.bfloat16(2.0))
    H1 = jnp.pad(A1[:, perm1[:k1]], ((0, 0), (0, k1p - k1)))
    A2 = _mm(G1, H1, zero_diag=True, out_dtype=jnp.bfloat16)

    xp1 = _pad2(xp1[:k1], k1p, hid)
    x3 = jax.nn.relu(_gcn_dense(A2, xp1, Wd2, bd2))

    perm2, inv2, xp2, _ = _pool(x3, p2, k1)
    G2 = jnp.pad(A2[perm2[:k2]], ((0, k2p - k2), (0, 0)))
    G2 = G2.at[jnp.arange(k2), perm2[:k2]].add(jnp.bfloat16(2.0))
    H2 = jnp.pad(A2[:, perm2[:k2]], ((0, 0), (0, k2p - k2)))
    A3 = _mm(G2, H2, zero_diag=True, out_dtype=jnp.bfloat16)

    xp2 = _pad2(xp2[:k2], k2p, hid)
    x4 = jax.nn.relu(_gcn_dense(A3, xp2, Wd3, bd3))

    up = jnp.zeros_like(x3).at[perm2[:k2]].set(x4[:k2])
    xq = x3 + up
    xq = jax.nn.relu(_gcn_dense(A2, xq, Wu0, bu0))
    up = jnp.zeros_like(x2).at[perm1[:k1]].set(xq[:k1])
    xq = x2 + up
    xq = jax.nn.relu(_gcn_dense(A1, xq, Wu1, bu1))
    up = jnp.zeros((n, hid), jnp.float32).at[perm0].set(xq[:k0], mode="drop")
    xq = x1 + up
    xq = gcn0(xq, Wu2, bu2)
    return jax.nn.log_softmax(xq, axis=1)


# speedup vs baseline: 3.3078x; 1.0002x over previous
import math
import functools

import jax
import jax.numpy as jnp
from jax import lax
from jax.experimental import pallas as pl
from jax.experimental.pallas import tpu as pltpu
from jax.experimental.pallas import tpu_sc as plsc

_DEPTH = 3
_RATIOS = (2000.0 / 2708.0, 0.5, 0.5)
_NT = 16
_RG = 96
_RH = 128


def _sc_build_gh(src, dst, inv, perm, k0p, np_):
    E = src.shape[0]
    assert E % (_NT * 16) == 0 and k0p % _RG == 0 and np_ % _RH == 0
    assert _RG * np_ == _RH * k0p
    ept = E // _NT
    ncg = k0p // _RG
    nch = np_ // _RH
    dump = _RG * np_
    import functools as _ft

    mesh = plsc.VectorSubcoreMesh(core_axis_name="c", subcore_axis_name="s")

    @_ft.partial(
        pl.kernel,
        mesh=mesh,
        out_type=(
            jax.ShapeDtypeStruct((k0p * np_,), jnp.float32),
            jax.ShapeDtypeStruct((np_ * k0p,), jnp.float32),
        ),
        scratch_types=[
            pltpu.VMEM((ept,), jnp.int32),
            pltpu.VMEM((ept,), jnp.int32),
            pltpu.VMEM((np_,), jnp.int32),
            pltpu.VMEM((k0p,), jnp.int32),
            pltpu.VMEM((np_,), jnp.float32),
            pltpu.VMEM((16,), jnp.float32),
            pltpu.VMEM((ept + 16,), jnp.int32),
            pltpu.VMEM((ept,), jnp.int32),
            pltpu.VMEM_SHARED((_RG * np_ + 16,), jnp.float32),
        ],
        compiler_params=pltpu.CompilerParams(needs_layout_passes=False),
    )
    def built(src_h, dst_h, inv_h, perm_h, zeros_h, ones_h, gp_h, h0_h,
              src_v, dst_v, inv_v, perm_v, zeros_v, ones_v, list_v, flat_v,
              spm):
        core = lax.axis_index("c")
        tile = lax.axis_index("s")
        pltpu.sync_copy(src_h.at[pl.ds(tile * ept, ept)], src_v)
        pltpu.sync_copy(dst_h.at[pl.ds(tile * ept, ept)], dst_v)
        pltpu.sync_copy(inv_h, inv_v)
        pltpu.sync_copy(perm_h, perm_v)
        pltpu.sync_copy(zeros_h, zeros_v)
        pltpu.sync_copy(ones_h, ones_v)
        iota16 = jnp.arange(16, dtype=jnp.int32)

        def one_matrix(nchunks, rows, cols, out_ref, rowinv, diag):
            srows = rows // _NT

            def pre(g, cy):
                s16 = src_v[pl.ds(g * 16, 16)]
                d16 = dst_v[pl.ds(g * 16, 16)]
                if rowinv:
                    row = plsc.load_gather(inv_v, [s16])
                    col = d16
                    ok = (row >= 0) & (s16 != d16)
                else:
                    row = s16
                    col = plsc.load_gather(inv_v, [d16])
                    ok = (col >= 0) & (s16 != d16)
                flat_v[pl.ds(g * 16, 16)] = jnp.where(ok, row * cols + col,
                                                      -1)
                return cy
            lax.fori_loop(0, ept // 16, pre, 0)

            def chunk_body(c, carry):
                @pl.when(c % 2 == core)
                def _chunk():
                    row0 = c * rows

                    def zrow(r, cy):
                        base = (tile * srows + r) * cols
                        pltpu.sync_copy(zeros_v.at[pl.ds(0, cols)],
                                        spm.at[pl.ds(base, cols)])
                        return cy
                    lax.fori_loop(0, srows, zrow, 0)

                    @pl.when(tile == 0)
                    def _zdump():
                        pltpu.sync_copy(zeros_v.at[pl.ds(0, 16)],
                                        spm.at[pl.ds(_RG * np_, 16)])
                    plsc.subcore_barrier()

                    if diag:
                        @pl.when(tile < rows // 16)
                        def _diag():
                            a16 = row0 + tile * 16 + iota16
                            p16 = plsc.load_gather(perm_v, [a16])
                            fl = jnp.where(p16 >= 0,
                                           (a16 - row0) * cols + p16, dump)
                            pltpu.sync_copy(ones_v, spm.at[fl], add=True)
                            pltpu.sync_copy(ones_v, spm.at[fl], add=True)

                    lo = row0 * cols
                    hi = lo + rows * cols

                    def scan(g, cnt):
                        f16 = flat_v[pl.ds(g * 16, 16)]
                        m = (f16 >= lo) & (f16 < hi)
                        plsc.store_compressed(list_v.at[pl.ds(cnt, 16)],
                                              f16 - lo, mask=m)
                        return cnt + jnp.sum(m.astype(jnp.int32))
                    cnt = lax.fori_loop(0, ept // 16, scan, 0)
                    list_v[pl.ds(cnt, 16)] = jnp.full((16,), dump, jnp.int32)

                    def scat(j, cy):
                        idx16 = list_v[pl.ds(j * 16, 16)]
                        pltpu.sync_copy(ones_v, spm.at[idx16], add=True)
                        return cy
                    lax.fori_loop(0, (cnt + 15) // 16, scat, 0)
                    plsc.subcore_barrier()

                    sz = srows * cols
                    pltpu.sync_copy(
                        spm.at[pl.ds(tile * sz, sz)],
                        out_ref.at[pl.ds(row0 * cols + tile * sz, sz)])
                return carry
            lax.fori_loop(0, nchunks, chunk_body, 0)

        one_matrix(ncg, _RG, np_, gp_h, rowinv=True, diag=True)
        one_matrix(nch, _RH, k0p, h0_h, rowinv=False, diag=False)

    zeros = jnp.zeros((np_,), jnp.float32)
    ones = jnp.ones((16,), jnp.float32)
    gp, h0 = built(src, dst, inv, perm, zeros, ones)
    return gp.reshape(k0p, np_), h0.reshape(np_, k0p)


def _sc_deg(srcp, dstp, E, np_):
    EP = srcp.shape[0]
    ept = EP // 32
    assert ept % 16 == 0
    mesh = plsc.VectorSubcoreMesh(core_axis_name="c", subcore_axis_name="s")

    @functools.partial(
        pl.kernel,
        mesh=mesh,
        out_type=jax.ShapeDtypeStruct((2, 2 * np_), jnp.float32),
        scratch_types=[
            pltpu.VMEM((ept,), jnp.int32),
            pltpu.VMEM((ept,), jnp.int32),
            pltpu.VMEM((np_,), jnp.float32),
            pltpu.VMEM((16,), jnp.float32),
            pltpu.VMEM_SHARED((2 * np_ + 16,), jnp.float32),
        ],
        compiler_params=pltpu.CompilerParams(needs_layout_passes=False),
    )
    def built(src_h, dst_h, zeros_h, ones_h, out_h, src_v, dst_v, zeros_v,
              ones_v, spm):
        core = lax.axis_index("c")
        tile = lax.axis_index("s")
        base = (core * _NT + tile) * ept
        pltpu.sync_copy(src_h.at[pl.ds(base, ept)], src_v)
        pltpu.sync_copy(dst_h.at[pl.ds(base, ept)], dst_v)
        pltpu.sync_copy(zeros_h, zeros_v)
        pltpu.sync_copy(ones_h, ones_v)
        iota16 = jnp.arange(16, dtype=jnp.int32)
        dump = 2 * np_

        @pl.when(tile < 2)
        def _z():
            pltpu.sync_copy(zeros_v, spm.at[pl.ds(tile * np_, np_)])

        @pl.when(tile == 2)
        def _z2():
            pltpu.sync_copy(zeros_v.at[pl.ds(0, 16)], spm.at[pl.ds(dump, 16)])
        plsc.subcore_barrier()

        def scan(g, cy):
            s16 = src_v[pl.ds(g * 16, 16)]
            d16 = dst_v[pl.ds(g * 16, 16)]
            mv = base + g * 16 + iota16 < E
            pltpu.sync_copy(ones_v, spm.at[jnp.where(mv, s16, dump)], add=True)
            selfm = mv & (s16 == d16)
            pltpu.sync_copy(ones_v, spm.at[jnp.where(selfm, np_ + s16, dump)],
                            add=True)
            return cy
        lax.fori_loop(0, ept // 16, scan, 0)
        plsc.subcore_barrier()

        @pl.when(tile < 2)
        def _wb():
            pltpu.sync_copy(spm.at[pl.ds(tile * np_, np_)],
                            out_h.at[core].at[pl.ds(tile * np_, np_)])

    zeros = jnp.zeros((np_,), jnp.float32)
    ones = jnp.ones((16,), jnp.float32)
    out = built(srcp, dstp, zeros, ones)
    comb = out[0] + out[1]
    return comb[:np_], comb[np_:]


def _ceil_to(v, m):
    return ((v + m - 1) // m) * m


def _pick(dim, cands):
    for c in cands:
        if dim % c == 0:
            return c
    raise AssertionError(dim)


def _mm(a, b, *, zero_diag=False, out_dtype=jnp.float32, split_b=False,
        bn_cap=4096):
    M, K = a.shape
    K2, N = b.shape
    assert K == K2, (a.shape, b.shape)
    bm = _pick(M, (1920, 1536, 768, 512, 256, 128))
    bn = _pick(N, (1920, 1536, 768, 512, 256, 128))
    bn = min(bn, bn_cap)
    bk = _pick(K, (512, 768, 256, 128))
    nk = K // bk

    def kern(a_ref, b_ref, o_ref, acc_ref):
        @pl.when(pl.program_id(2) == 0)
        def _():
            acc_ref[...] = jnp.zeros_like(acc_ref)

        av = a_ref[...].astype(jnp.bfloat16)
        bv = b_ref[...]
        if split_b:
            bhi = bv.astype(jnp.bfloat16)
            blo = (bv - bhi.astype(jnp.float32)).astype(jnp.bfloat16)
            acc_ref[...] += (jnp.dot(av, bhi, preferred_element_type=jnp.float32)
                             + jnp.dot(av, blo, preferred_element_type=jnp.float32))
        else:
            acc_ref[...] += jnp.dot(av, bv.astype(jnp.bfloat16),
                                    preferred_element_type=jnp.float32)

        if zero_diag:
            gi = pl.program_id(0) * bm + lax.broadcasted_iota(jnp.int32, (bm, bn), 0)
            gj = pl.program_id(1) * bn + lax.broadcasted_iota(jnp.int32, (bm, bn), 1)
            diag = gi == gj
        else:
            diag = None
        last = pl.program_id(2) == nk - 1

        @pl.when(last)
        def _():
            r = acc_ref[...]
            if diag is not None:
                r = jnp.where(diag, 0.0, r)
            o_ref[...] = r.astype(out_dtype)

    return pl.pallas_call(
        kern,
        grid=(M // bm, N // bn, nk),
        in_specs=[
            pl.BlockSpec((bm, bk), lambda i, j, k: (i, k)),
            pl.BlockSpec((bk, bn), lambda i, j, k: (k, j)),
        ],
        out_specs=pl.BlockSpec((bm, bn), lambda i, j, k: (i, j)),
        out_shape=jax.ShapeDtypeStruct((M, N), out_dtype),
        scratch_shapes=[pltpu.VMEM((bm, bn), jnp.float32)],
        compiler_params=pltpu.CompilerParams(
            dimension_semantics=("parallel", "parallel", "arbitrary")),
    )(a, b)


def _mm_f32(a, b):
    M, K = a.shape
    _, N = b.shape
    bm = _pick(M, (512, 256, 128))
    bn = _pick(N, (256, 128))
    bk = _pick(K, (512, 256, 128, 32))
    nk = K // bk

    def kern(a_ref, b_ref, o_ref, acc_ref):
        @pl.when(pl.program_id(2) == 0)
        def _():
            acc_ref[...] = jnp.zeros_like(acc_ref)

        acc_ref[...] += jnp.dot(a_ref[...], b_ref[...],
                                preferred_element_type=jnp.float32,
                                precision=lax.Precision.HIGHEST)
        last = pl.program_id(2) == nk - 1

        @pl.when(last)
        def _():
            o_ref[...] = acc_ref[...]

    return pl.pallas_call(
        kern,
        grid=(M // bm, N // bn, nk),
        in_specs=[
            pl.BlockSpec((bm, bk), lambda i, j, k: (i, k)),
            pl.BlockSpec((bk, bn), lambda i, j, k: (k, j)),
        ],
        out_specs=pl.BlockSpec((bm, bn), lambda i, j, k: (i, j)),
        out_shape=jax.ShapeDtypeStruct((M, N), jnp.float32),
        scratch_shapes=[pltpu.VMEM((bm, bn), jnp.float32)],
        compiler_params=pltpu.CompilerParams(
            dimension_semantics=("parallel", "parallel", "arbitrary")),
    )(a, b)


def _dis_from_rowsum(Al):
    M, N = Al.shape
    bm = _pick(M, (768, 512, 256, 128))
    bn = _pick(N, (768, 512, 256, 128))
    nj = N // bn

    def kern(a_ref, o_ref, acc_ref):
        @pl.when(pl.program_id(1) == 0)
        def _():
            acc_ref[...] = jnp.zeros_like(acc_ref)

        acc_ref[...] += jnp.sum(a_ref[...].astype(jnp.float32), axis=1,
                                keepdims=True)

        @pl.when(pl.program_id(1) == nj - 1)
        def _():
            o_ref[...] = lax.rsqrt(acc_ref[...] + 2.0)

    return pl.pallas_call(
        kern,
        grid=(M // bm, nj),
        in_specs=[pl.BlockSpec((bm, bn), lambda i, j: (i, j))],
        out_specs=pl.BlockSpec((bm, 1), lambda i, j: (i, 0)),
        out_shape=jax.ShapeDtypeStruct((M, 1), jnp.float32),
        scratch_shapes=[pltpu.VMEM((bm, 1), jnp.float32)],
        compiler_params=pltpu.CompilerParams(
            dimension_semantics=("parallel", "arbitrary")),
    )(Al)


def _pad2(a, m, n):
    return jnp.pad(a, ((0, m - a.shape[0]), (0, n - a.shape[1])))


def _mm_feat(a, w):
    M = a.shape[0]
    K = _ceil_to(a.shape[1], 32)
    N = w.shape[1]
    Np = _ceil_to(N, 128)
    ap = _pad2(a, M, K)
    wp = _pad2(w, K, Np)
    r = _mm_f32(ap, wp)
    return r[:, :N]


def _gcn_dense(Al, xl, W, b):
    dis = _dis_from_rowsum(Al)
    xw = _mm_feat(xl, W)
    u = dis * xw
    kp = Al.shape[0]
    up = _pad2(u, kp, 128)
    m = _mm(Al, up, split_b=True)[:, : xw.shape[1]]
    out = dis * m + (2.0 * dis * dis) * xw + b[None, :]
    return out


_RATIOS_K = {}


def _pool(xl, w, nreal):
    nl = xl.shape[0]
    k = int(math.ceil(_RATIOS_K[nreal]))
    score = jnp.tanh((xl @ w) / jnp.linalg.norm(w))
    score = jnp.where(jnp.arange(nl) < nreal, score, -2.0)
    _, permd = lax.top_k(score, k)
    perm = jnp.sort(permd)
    sv = score[perm]
    xp = xl[perm] * sv[:, None]
    inv = jnp.full((nl,), -1, jnp.int32).at[perm].set(jnp.arange(k, dtype=jnp.int32))
    return perm, inv, xp, k


def kernel(x, edge_index, Wd0, bd0, Wd1, bd1, Wd2, bd2, Wd3, bd3,
           p0, p1, p2, Wu0, bu0, Wu1, bu1, Wu2, bu2):
    n = x.shape[0]
    hid = Wd0.shape[1]
    src, dst = edge_index[0], edge_index[1]
    selfm = src == dst

    k0 = int(math.ceil(_RATIOS[0] * n))
    k1 = int(math.ceil(_RATIOS[1] * k0))
    k2 = int(math.ceil(_RATIOS[2] * k1))
    np_ = _ceil_to(n, 512)
    k0p = _ceil_to(k0, 768 if k0 > 768 else 256)
    k1p = _ceil_to(k1, 768 if k1 > 768 else 256)
    k2p = _ceil_to(k2, 768 if k2 > 768 else 256)
    _RATIOS_K[n] = _RATIOS[0] * n
    _RATIOS_K[k0] = _RATIOS[1] * k0
    _RATIOS_K[k1] = _RATIOS[2] * k1

    E = src.shape[0]
    big = n >= 1024
    if big:
        EP2 = _ceil_to(E, 512)
        srcp = jnp.pad(src, (0, EP2 - E))
        dstp = jnp.pad(dst, (0, EP2 - E))
        rowsum0, c0 = _sc_deg(srcp, dstp, E, np_)
        rowsum0, c0 = rowsum0[:n], c0[:n]
    else:
        ones_e = jnp.ones(src.shape, jnp.float32)
        rowsum0 = jnp.zeros((n,), jnp.float32).at[src].add(ones_e)
        c0 = jnp.zeros((n,), jnp.float32).at[src].add(
            jnp.where(selfm, 1.0, 0.0))
    a2ii = jnp.where(c0 > 0, c0, 2.0)
    deg0 = rowsum0 - c0 + a2ii
    dis0 = lax.rsqrt(deg0)

    def msgpass(u):
        return jnp.zeros_like(u).at[src].add(
            jnp.where(selfm, 0.0, 1.0)[:, None] * u[dst])

    def gcn0(xl, W, b):
        xw = _mm_feat(_pad2(xl, np_, xl.shape[1]), W)[:n]
        if xl.shape[1] <= W.shape[1]:
            m32 = msgpass(dis0[:, None] * xl)
            msg = _mm_feat(_pad2(m32, np_, m32.shape[1]), W)[:n]
        else:
            msg = msgpass(dis0[:, None] * xw)
        return dis0[:, None] * msg + (a2ii * dis0 * dis0)[:, None] * xw + b[None, :]

    x1 = jax.nn.relu(gcn0(x, Wd0, bd0))

    x1p = _pad2(x1, np_, hid)
    perm0, inv0, xp0, _ = _pool(x1p, p0, n)
    if n >= 1024:
        perm0p = jnp.pad(perm0, (0, k0p - k0), constant_values=-1)
        Gp, H0 = _sc_build_gh(src, dst, inv0, perm0p, k0p, np_)
    else:
        one = jnp.ones((), jnp.float32)
        gi = jnp.where((inv0[src] >= 0) & (~selfm), inv0[src], k0p)
        Gp = jnp.zeros((k0p, np_), jnp.float32).at[gi, dst].add(one, mode="drop")
        Gp = Gp.at[jnp.arange(k0), perm0[:k0]].add(2.0, mode="drop")
        hi = jnp.where((inv0[dst] >= 0) & (~selfm), inv0[dst], k0p)
        H0 = jnp.zeros((np_, k0p), jnp.float32).at[src, hi].add(one, mode="drop")
    A1 = _mm(Gp, H0, zero_diag=True, out_dtype=jnp.bfloat16)

    xp0 = _pad2(xp0[:k0], k0p, hid)
    x2 = jax.nn.relu(_gcn_dense(A1, xp0, Wd1, bd1))

    perm1, inv1, xp1, _ = _pool(x2, p1, k0)
    G1 = jnp.pad(A1[perm1[:k1]], ((0, k1p - k1), (0, 0)))
    G1 = G1.at[jnp.arange(k1), perm1[:k1]].add(jnp.bfloat16(2.0))
    H1 = jnp.pad(A1[:, perm1[:k1]], ((0, 0), (0, k1p - k1)))
    A2 = _mm(G1, H1, zero_diag=True, out_dtype=jnp.bfloat16)

    xp1 = _pad2(xp1[:k1], k1p, hid)
    x3 = jax.nn.relu(_gcn_dense(A2, xp1, Wd2, bd2))

    perm2, inv2, xp2, _ = _pool(x3, p2, k1)
    G2 = jnp.pad(A2[perm2[:k2]], ((0, k2p - k2), (0, 0)))
    G2 = G2.at[jnp.arange(k2), perm2[:k2]].add(jnp.bfloat16(2.0))
    H2 = jnp.pad(A2[:, perm2[:k2]], ((0, 0), (0, k2p - k2)))
    A3 = _mm(G2, H2, zero_diag=True, out_dtype=jnp.bfloat16)

    xp2 = _pad2(xp2[:k2], k2p, hid)
    x4 = jax.nn.relu(_gcn_dense(A3, xp2, Wd3, bd3))

    up = jnp.zeros_like(x3).at[perm2[:k2]].set(x4[:k2])
    xq = x3 + up
    xq = jax.nn.relu(_gcn_dense(A2, xq, Wu0, bu0))
    up = jnp.zeros_like(x2).at[perm1[:k1]].set(xq[:k1])
    xq = x2 + up
    xq = jax.nn.relu(_gcn_dense(A1, xq, Wu1, bu1))
    up = jnp.zeros((n, hid), jnp.float32).at[perm0].set(xq[:k0], mode="drop")
    xq = x1 + up
    xq = gcn0(xq, Wu2, bu2)
    return jax.nn.log_softmax(xq, axis=1)
